# Initial kernel scaffold; baseline (speedup 1.0000x reference)
import jax
import jax.numpy as jnp
from jax.experimental import pallas as pl


def kernel(pointcloud, params):
    raise NotImplementedError("write your pallas kernel here")


# speedup vs baseline: 9.1833x; 9.1833x over previous
import functools
import numpy as np
import jax
import jax.numpy as jnp
from jax.experimental import pallas as pl
from jax.experimental.pallas import tpu as pltpu

_INTERPRET = False

_BN_INV = 1.0 / np.sqrt(1.0 + 1e-5)
F32 = jnp.float32
BIG = 3.0e7


def _iota(shape, dim):
    return jax.lax.broadcasted_iota(jnp.int32, shape, dim).astype(F32)


def _b16(x):
    return x.astype(jnp.bfloat16).astype(F32)


def _geom1_body(n, npoint, radii, nsamples, xyzT_ref, nx_ref, i0_ref, i1_ref):
    nr = n // 2048
    px = xyzT_ref[0, 0:1, :]
    py = xyzT_ref[0, 1:2, :]
    pz = xyzT_ref[0, 2:3, :]
    px8 = px.reshape(nr, 2048)
    py8 = py.reshape(nr, 2048)
    pz8 = pz.reshape(nr, 2048)
    n8 = _iota((nr, 2048), 0) * 2048.0 + _iota((nr, 2048), 1)
    riota = _iota((npoint, 1), 0)

    def fps_step(s, carry):
        dists, far, nxx, nxy, nxz = carry
        sel = (n8 == far)
        cx = jnp.sum(jnp.where(sel, px8, 0.0))
        cy = jnp.sum(jnp.where(sel, py8, 0.0))
        cz = jnp.sum(jnp.where(sel, pz8, 0.0))
        srow = (riota == s.astype(F32))
        nxx = jnp.where(srow, cx, nxx)
        nxy = jnp.where(srow, cy, nxy)
        nxz = jnp.where(srow, cz, nxz)
        dx = px8 - cx
        dy = py8 - cy
        dz = pz8 - cz
        d = dx * dx + dy * dy + dz * dz
        dists = jnp.minimum(dists, d)
        m = jnp.max(dists)
        far = jnp.min(jnp.where(dists == m, n8, BIG))
        return dists, far, nxx, nxy, nxz

    init = (jnp.full((nr, 2048), 1e10, F32), jnp.float32(0.0),
            jnp.zeros((npoint, 1), F32), jnp.zeros((npoint, 1), F32),
            jnp.zeros((npoint, 1), F32))
    _, _, nxx, nxy, nxz = jax.lax.fori_loop(0, npoint, fps_step, init)
    nx_ref[0] = jnp.concatenate([nxx, nxy, nxz], axis=1)

    aa = nxx * nxx + nxy * nxy + nxz * nxz
    bb = px * px + py * py + pz * pz
    ab = (_b16(nxx) * _b16(px) + _b16(nxy) * _b16(py)) + _b16(nxz) * _b16(pz)
    sqd = jnp.maximum(aa + bb - 2.0 * ab, 0.0)
    niota = _iota((1, n), 1)
    nf = float(n)
    for (radius, k, out_ref) in ((radii[0], nsamples[0], i0_ref),
                                 (radii[1], nsamples[1], i1_ref)):
        dsel = jnp.where(sqd < radius * radius, niota, nf)
        cols = []
        for _ in range(k):
            mk = jnp.min(dsel, axis=1, keepdims=True)
            cols.append(mk)
            dsel = jnp.where(dsel == mk, BIG, dsel)
        idx = jnp.concatenate(cols, axis=1)
        first = idx[:, 0:1]
        idx = jnp.where(idx >= nf, first, idx)
        idx = jnp.where(idx >= nf, 0.0, idx)
        out_ref[0] = idx.astype(jnp.int32)


def _geom1(xyzT, n, b, npoint, radii, nsamples):
    body = functools.partial(_geom1_body, n, npoint, radii, nsamples)
    return pl.pallas_call(
        body,
        grid=(b,),
        in_specs=[pl.BlockSpec((1, 3, n), lambda i: (i, 0, 0))],
        out_specs=[
            pl.BlockSpec((1, npoint, 3), lambda i: (i, 0, 0)),
            pl.BlockSpec((1, npoint, nsamples[0]), lambda i: (i, 0, 0)),
            pl.BlockSpec((1, npoint, nsamples[1]), lambda i: (i, 0, 0)),
        ],
        out_shape=[
            jax.ShapeDtypeStruct((b, npoint, 3), F32),
            jax.ShapeDtypeStruct((b, npoint, nsamples[0]), jnp.int32),
            jax.ShapeDtypeStruct((b, npoint, nsamples[1]), jnp.int32),
        ],
        interpret=_INTERPRET,
    )(xyzT)


def _mlp_rows(h, layers):
    for (w, bias) in layers:
        h = jnp.maximum(jnp.dot(h, w, preferred_element_type=F32) + bias, 0.0)
    return h


def _group1_body(n, npoint, nsamples, nlayers, args):
    pc_ref, nx_ref, i0_ref, i1_ref = args[:4]
    wrefs = args[4:4 + 2 * nlayers[0] + 2 * nlayers[1]]
    f_ref = args[4 + 2 * nlayers[0] + 2 * nlayers[1]]
    g0_ref, g1_ref = args[-2:]
    k0, k1 = nsamples

    def gather(s, _):
        for k in range(k0):
            i = i0_ref[0, s, k]
            g0_ref[pl.ds(k * npoint + s, 1), :] = pc_ref[0, pl.ds(i, 1), :]
        for k in range(k1):
            i = i1_ref[0, s, k]
            g1_ref[pl.ds(k * npoint + s, 1), :] = pc_ref[0, pl.ds(i, 1), :]
        return 0

    jax.lax.fori_loop(0, npoint, gather, 0)
    nx = nx_ref[0]
    outs = []
    woff = 0
    for (k, g_ref, nl) in ((k0, g0_ref, nlayers[0]), (k1, g1_ref, nlayers[1])):
        layers = [(wrefs[woff + 2 * j][...], wrefs[woff + 2 * j + 1][...])
                  for j in range(nl)]
        woff += 2 * nl
        g = g_ref[...]
        cent = jnp.concatenate([nx] * k, axis=0)
        h = jnp.concatenate([g[:, 0:3] - cent, g[:, 3:]], axis=1)
        h = _mlp_rows(h, layers)
        o = h[0:npoint]
        for j in range(1, k):
            o = jnp.maximum(o, h[j * npoint:(j + 1) * npoint])
        outs.append(o)
    f_ref[0] = jnp.concatenate(outs, axis=1)


def _group1(pointcloud, nx, i0, i1, sa_layers, n, b, npoint, nsamples):
    ch = pointcloud.shape[-1]
    nlayers = (len(sa_layers[0]), len(sa_layers[1]))
    wargs, wspecs = [], []
    for scale in sa_layers:
        for (w, bias) in scale:
            wargs += [w, bias]
            wspecs += [pl.BlockSpec(w.shape, lambda i: (0, 0)),
                       pl.BlockSpec(bias.shape, lambda i: (0, 0))]
    cout = sum(int(s[-1][0].shape[1]) for s in sa_layers)
    body = functools.partial(_group1_body, n, npoint, nsamples, nlayers)

    def wrapped(*refs):
        body(refs)

    return pl.pallas_call(
        wrapped,
        grid=(b,),
        in_specs=[
            pl.BlockSpec((1, n, ch), lambda i: (i, 0, 0)),
            pl.BlockSpec((1, npoint, 3), lambda i: (i, 0, 0)),
            pl.BlockSpec((1, npoint, nsamples[0]), lambda i: (i, 0, 0),
                         memory_space=pltpu.SMEM),
            pl.BlockSpec((1, npoint, nsamples[1]), lambda i: (i, 0, 0),
                         memory_space=pltpu.SMEM),
        ] + wspecs,
        out_specs=[pl.BlockSpec((1, npoint, cout), lambda i: (i, 0, 0))],
        out_shape=[jax.ShapeDtypeStruct((b, npoint, cout), F32)],
        scratch_shapes=[pltpu.VMEM((nsamples[0] * npoint, ch), F32),
                        pltpu.VMEM((nsamples[1] * npoint, ch), F32)],
        interpret=_INTERPRET,
    )(pointcloud, nx, i0, i1, *wargs)[0]


def _fps_small(xyz, npoint):
    p = xyz.shape[0]
    px = xyz[:, 0:1].reshape(1, p)
    py = xyz[:, 1:2].reshape(1, p)
    pz = xyz[:, 2:3].reshape(1, p)
    niota = _iota((1, p), 1)
    riota = _iota((npoint, 1), 0)

    def step(s, carry):
        dists, far, nxx, nxy, nxz = carry
        sel = (niota == far)
        cx = jnp.sum(jnp.where(sel, px, 0.0))
        cy = jnp.sum(jnp.where(sel, py, 0.0))
        cz = jnp.sum(jnp.where(sel, pz, 0.0))
        srow = (riota == s.astype(F32))
        nxx = jnp.where(srow, cx, nxx)
        nxy = jnp.where(srow, cy, nxy)
        nxz = jnp.where(srow, cz, nxz)
        dx = px - cx
        dy = py - cy
        dz = pz - cz
        d = dx * dx + dy * dy + dz * dz
        dists = jnp.minimum(dists, d)
        m = jnp.max(dists)
        far = jnp.min(jnp.where(dists == m, niota, BIG))
        return dists, far, nxx, nxy, nxz

    init = (jnp.full((1, p), 1e10, F32), jnp.float32(0.0),
            jnp.zeros((npoint, 1), F32), jnp.zeros((npoint, 1), F32),
            jnp.zeros((npoint, 1), F32))
    _, _, nxx, nxy, nxz = jax.lax.fori_loop(0, npoint, step, init)
    return jnp.concatenate([nxx, nxy, nxz], axis=1)


def _sqd_small(a, bpts):
    p = bpts.shape[0]
    ax, ay, az = a[:, 0:1], a[:, 1:2], a[:, 2:3]
    bx = bpts[:, 0:1].reshape(1, p)
    by = bpts[:, 1:2].reshape(1, p)
    bz = bpts[:, 2:3].reshape(1, p)
    aa = ax * ax + ay * ay + az * az
    bb = bx * bx + by * by + bz * bz
    ab = (_b16(ax) * _b16(bx) + _b16(ay) * _b16(by)) + _b16(az) * _b16(bz)
    return jnp.maximum(aa + bb - 2.0 * ab, 0.0)


def _ball_small(sqd, radius, k):
    s, p = sqd.shape
    niota = _iota((1, p), 1)
    pf = float(p)
    dsel = jnp.where(sqd < radius * radius, niota, pf)
    cols = []
    for _ in range(k):
        mk = jnp.min(dsel, axis=1, keepdims=True)
        cols.append(mk)
        dsel = jnp.where(dsel == mk, BIG, dsel)
    idx = jnp.concatenate(cols, axis=1)
    first = idx[:, 0:1]
    idx = jnp.where(idx >= pf, first, idx)
    idx = jnp.where(idx >= pf, 0.0, idx)
    return idx


def _gather_oh(idxcol, table):
    p = table.shape[0]
    oh = (_iota((1, p), 1) == idxcol).astype(F32)
    return jnp.dot(oh, table, preferred_element_type=F32)


def _sa_small(xyz, feats, npoint, radii, nsamples, scales):
    p = xyz.shape[0]
    nx = _fps_small(xyz, npoint)
    table = jnp.concatenate([xyz, feats], axis=1)
    sqd = _sqd_small(nx, xyz)
    outs = []
    for radius, k, layers in zip(radii, nsamples, scales):
        idx = _ball_small(sqd, radius, k)
        hs = []
        for j in range(k):
            rows = _gather_oh(idx[:, j:j + 1], table)
            hs.append(jnp.concatenate([rows[:, 0:3] - nx, rows[:, 3:]], axis=1))
        h = jnp.concatenate(hs, axis=0)
        h = _mlp_rows(h, layers)
        o = h[0:npoint]
        for j in range(1, k):
            o = jnp.maximum(o, h[j * npoint:(j + 1) * npoint])
        outs.append(o)
    return nx, jnp.concatenate(outs, axis=1)


def _interp3(unknown, known, kn_f):
    u = unknown.shape[0]
    p = known.shape[0]
    sqd = _sqd_small(unknown, known)
    kiota = _iota((1, p), 1)
    d = sqd
    wmat = jnp.zeros((u, p), F32)
    for _ in range(3):
        m = jnp.min(d, axis=1, keepdims=True)
        idxk = jnp.min(jnp.where(d == m, kiota, BIG), axis=1, keepdims=True)
        oh = (kiota == idxk)
        w = 1.0 / (jnp.sqrt(jnp.maximum(m, 0.0)) + 1e-8)
        wmat = wmat + jnp.where(oh, w, 0.0)
        d = jnp.where(oh, BIG, d)
    wmat = wmat / jnp.sum(wmat, axis=1, keepdims=True)
    return jnp.dot(wmat, kn_f, preferred_element_type=F32)


def _mid_body(cfg2, cfg3, nl, args):
    xyz1_ref, f1_ref = args[:2]
    wrefs = args[2:2 + 2 * sum(nl)]
    out_ref = args[2 + 2 * sum(nl)]

    def take(off, count):
        return [(wrefs[off + 2 * j][...], wrefs[off + 2 * j + 1][...])
                for j in range(count)]

    o = 0
    sa2_l0 = take(o, nl[0]); o += 2 * nl[0]
    sa2_l1 = take(o, nl[1]); o += 2 * nl[1]
    sa3_l0 = take(o, nl[2]); o += 2 * nl[2]
    sa3_l1 = take(o, nl[3]); o += 2 * nl[3]
    fp2_l = take(o, nl[4]); o += 2 * nl[4]
    fp1_l = take(o, nl[5]); o += 2 * nl[5]

    xyz1 = xyz1_ref[0]
    f1 = f1_ref[0]
    np2, radii2, ns2 = cfg2
    np3, radii3, ns3 = cfg3
    xyz2, f2 = _sa_small(xyz1, f1, np2, radii2, ns2, (sa2_l0, sa2_l1))
    xyz3, f3 = _sa_small(xyz2, f2, np3, radii3, ns3, (sa3_l0, sa3_l1))
    h = jnp.concatenate([_interp3(xyz2, xyz3, f3), f2], axis=1)
    f2p = _mlp_rows(h, fp2_l)
    h = jnp.concatenate([_interp3(xyz1, xyz2, f2p), f1], axis=1)
    out_ref[0] = _mlp_rows(h, fp1_l)


def _mid(nx1, f1, wlists, cfg2, cfg3, b):
    npoint = nx1.shape[1]
    c1 = f1.shape[-1]
    nl = tuple(len(l) for l in wlists)
    wargs, wspecs = [], []
    for lst in wlists:
        for (w, bias) in lst:
            wargs += [w, bias]
            wspecs += [pl.BlockSpec(w.shape, lambda i: (0, 0)),
                       pl.BlockSpec(bias.shape, lambda i: (0, 0))]
    cout = int(wlists[-1][-1][0].shape[1])
    body = functools.partial(_mid_body, cfg2, cfg3, nl)

    def wrapped(*refs):
        body(refs)

    return pl.pallas_call(
        wrapped,
        grid=(b,),
        in_specs=[
            pl.BlockSpec((1, npoint, 3), lambda i: (i, 0, 0)),
            pl.BlockSpec((1, npoint, c1), lambda i: (i, 0, 0)),
        ] + wspecs,
        out_specs=[pl.BlockSpec((1, npoint, cout), lambda i: (i, 0, 0))],
        out_shape=[jax.ShapeDtypeStruct((b, npoint, cout), F32)],
        interpret=_INTERPRET,
    )(nx1, f1, *wargs)[0]


def _head_body(nl, args):
    pc_ref, nx_ref, f1p_ref = args[:3]
    wrefs = args[3:3 + 2 * nl]
    out_ref = args[3 + 2 * nl]
    layers = [(wrefs[2 * j][...], wrefs[2 * j + 1][...]) for j in range(nl)]
    pc = pc_ref[0]
    xt = pc[:, 0:3]
    ft = pc[:, 3:]
    interp = _interp3(xt, nx_ref[0], f1p_ref[0])
    h = jnp.concatenate([interp, ft], axis=1)
    nrelu = nl - 1
    for j in range(nrelu):
        w, bias = layers[j]
        h = jnp.maximum(jnp.dot(h, w, preferred_element_type=F32) + bias, 0.0)
    w, bias = layers[nrelu]
    out_ref[0] = jnp.dot(h, w, preferred_element_type=F32) + bias


def _head(pointcloud, nx1, f1p, layers, b, n, tile):
    ch = pointcloud.shape[-1]
    npoint = nx1.shape[1]
    c1 = f1p.shape[-1]
    wargs, wspecs = [], []
    for (w, bias) in layers:
        wargs += [w, bias]
        wspecs += [pl.BlockSpec(w.shape, lambda i, t: (0, 0)),
                   pl.BlockSpec(bias.shape, lambda i, t: (0, 0))]
    ncls = int(layers[-1][0].shape[1])
    body = functools.partial(_head_body, len(layers))

    def wrapped(*refs):
        body(refs)

    return pl.pallas_call(
        wrapped,
        grid=(b, n // tile),
        in_specs=[
            pl.BlockSpec((1, tile, ch), lambda i, t: (i, t, 0)),
            pl.BlockSpec((1, npoint, 3), lambda i, t: (i, 0, 0)),
            pl.BlockSpec((1, npoint, c1), lambda i, t: (i, 0, 0)),
        ] + wspecs,
        out_specs=[pl.BlockSpec((1, tile, ncls), lambda i, t: (i, t, 0))],
        out_shape=[jax.ShapeDtypeStruct((b, n, ncls), F32)],
        interpret=_INTERPRET,
    )(pointcloud, nx1, f1p, *wargs)[0]


def _fold(layers):
    out = []
    for (w, g, bias) in layers:
        out.append(((w * (g * _BN_INV)[:, None]).T, bias.reshape(1, -1)))
    return out


def kernel(pointcloud, params):
    b, n, ch = pointcloud.shape
    xyzT = jnp.transpose(pointcloud[..., :3], (0, 2, 1))

    sa_cfg = [(128, (1.0, 3.0), (2, 8)),
              (128, (2.0, 4.0), (2, 8)),
              (64, (3.0, 6.0), (4, 8))]

    np1, radii1, ns1 = sa_cfg[0]
    nx1, i0, i1 = _geom1(xyzT, n, b, np1, radii1, ns1)

    sa1_layers = [_fold(s) for s in params['sa'][0]]
    f1 = _group1(pointcloud, nx1, i0, i1, sa1_layers, n, b, np1, ns1)

    wlists = ([_fold(s) for s in params['sa'][1]]
              + [_fold(s) for s in params['sa'][2]]
              + [_fold(params['fp'][2]), _fold(params['fp'][1])])
    f1p = _mid(nx1, f1, wlists, sa_cfg[1], sa_cfg[2], b)

    fc = params['fc']
    head_layers = _fold(params['fp'][0]) + [
        ((fc['W1'] * (fc['g1'] * _BN_INV)[:, None]).T, fc['b1'].reshape(1, -1)),
        (fc['W2'].T, fc['b2'].reshape(1, -1)),
        (fc['W3'].T, fc['b3'].reshape(1, -1)),
    ]
    return _head(pointcloud, nx1, f1p, head_layers, b, n, 2048)


# speedup vs baseline: 26.0629x; 2.8381x over previous
import functools
import numpy as np
import jax
import jax.numpy as jnp
from jax.experimental import pallas as pl
from jax.experimental.pallas import tpu as pltpu

_INTERPRET = False

_BN_INV = 1.0 / np.sqrt(1.0 + 1e-5)
F32 = jnp.float32
BIG = 3.0e7


def _iota(shape, dim):
    return jax.lax.broadcasted_iota(jnp.int32, shape, dim).astype(F32)


def _b16(x):
    return x.astype(jnp.bfloat16).astype(F32)


def _fps1_body(npoint, xyzT_ref, nx_ref):
    px = xyzT_ref[:, 0, :]
    py = xyzT_ref[:, 1, :]
    pz = xyzT_ref[:, 2, :]
    nxx, nxy, nxz = _fps_batched(px, py, pz, npoint)
    nx_ref[:, 0, :] = nxx
    nx_ref[:, 1, :] = nxy
    nx_ref[:, 2, :] = nxz


def _fps1(xyzT, n, b, npoint):
    body = functools.partial(_fps1_body, npoint)
    return pl.pallas_call(
        body,
        grid=(1,),
        in_specs=[pl.BlockSpec((b, 3, n), lambda i: (0, 0, 0))],
        out_specs=[pl.BlockSpec((b, 3, npoint), lambda i: (0, 0, 0))],
        out_shape=[jax.ShapeDtypeStruct((b, 3, npoint), F32)],
        interpret=_INTERPRET,
    )(xyzT)[0]


def _bq1_body(n, npoint, radii, nsamples, xyzT_ref, nxT_ref, i0_ref, i1_ref):
    px = xyzT_ref[0, 0:1, :]
    py = xyzT_ref[0, 1:2, :]
    pz = xyzT_ref[0, 2:3, :]
    nxx = nxT_ref[0, 0:1, :].reshape(npoint, 1)
    nxy = nxT_ref[0, 1:2, :].reshape(npoint, 1)
    nxz = nxT_ref[0, 2:3, :].reshape(npoint, 1)
    aa = nxx * nxx + nxy * nxy + nxz * nxz
    bb = px * px + py * py + pz * pz
    ab = (_b16(nxx) * _b16(px) + _b16(nxy) * _b16(py)) + _b16(nxz) * _b16(pz)
    sqd = jnp.maximum(aa + bb - 2.0 * ab, 0.0)
    niota = _iota((1, n), 1)
    nf = float(n)
    for (radius, k, out_ref) in ((radii[0], nsamples[0], i0_ref),
                                 (radii[1], nsamples[1], i1_ref)):
        dsel = jnp.where(sqd < radius * radius, niota, nf)
        cols = []
        for _ in range(k):
            mk = jnp.min(dsel, axis=1, keepdims=True)
            cols.append(mk)
            dsel = jnp.where(dsel == mk, BIG, dsel)
        idx = jnp.concatenate(cols, axis=1)
        first = idx[:, 0:1]
        idx = jnp.where(idx >= nf, first, idx)
        idx = jnp.where(idx >= nf, 0.0, idx)
        out_ref[0] = idx.astype(jnp.int32)


def _bq1(xyzT, nxT, n, b, npoint, radii, nsamples):
    body = functools.partial(_bq1_body, n, npoint, radii, nsamples)
    return pl.pallas_call(
        body,
        grid=(b,),
        in_specs=[
            pl.BlockSpec((1, 3, n), lambda i: (i, 0, 0)),
            pl.BlockSpec((1, 3, npoint), lambda i: (i, 0, 0)),
        ],
        out_specs=[
            pl.BlockSpec((1, npoint, nsamples[0]), lambda i: (i, 0, 0)),
            pl.BlockSpec((1, npoint, nsamples[1]), lambda i: (i, 0, 0)),
        ],
        out_shape=[
            jax.ShapeDtypeStruct((b, npoint, nsamples[0]), jnp.int32),
            jax.ShapeDtypeStruct((b, npoint, nsamples[1]), jnp.int32),
        ],
        interpret=_INTERPRET,
    )(xyzT, nxT)


def _mlp_rows(h, layers):
    for (w, bias) in layers:
        h = jnp.maximum(jnp.dot(h, w, preferred_element_type=F32) + bias, 0.0)
    return h


def _group1_body(n, npoint, nsamples, nlayers, args):
    pc_ref, nx_ref, i0_ref, i1_ref = args[:4]
    wrefs = args[4:4 + 2 * nlayers[0] + 2 * nlayers[1]]
    f_ref = args[4 + 2 * nlayers[0] + 2 * nlayers[1]]
    g0_ref, g1_ref = args[-2:]
    k0, k1 = nsamples

    def gather(s, _):
        for k in range(k0):
            i = i0_ref[0, s, k]
            g0_ref[pl.ds(k * npoint + s, 1), :] = pc_ref[0, pl.ds(i, 1), :]
        for k in range(k1):
            i = i1_ref[0, s, k]
            g1_ref[pl.ds(k * npoint + s, 1), :] = pc_ref[0, pl.ds(i, 1), :]
        return 0

    jax.lax.fori_loop(0, npoint, gather, 0)
    nx = nx_ref[0]
    outs = []
    woff = 0
    for (k, g_ref, nl) in ((k0, g0_ref, nlayers[0]), (k1, g1_ref, nlayers[1])):
        layers = [(wrefs[woff + 2 * j][...], wrefs[woff + 2 * j + 1][...])
                  for j in range(nl)]
        woff += 2 * nl
        g = g_ref[...]
        cent = jnp.concatenate([nx] * k, axis=0)
        h = jnp.concatenate([g[:, 0:3] - cent, g[:, 3:]], axis=1)
        h = _mlp_rows(h, layers)
        o = h[0:npoint]
        for j in range(1, k):
            o = jnp.maximum(o, h[j * npoint:(j + 1) * npoint])
        outs.append(o)
    f_ref[0] = jnp.concatenate(outs, axis=1)


def _group1(pointcloud, nx, i0, i1, sa_layers, n, b, npoint, nsamples):
    ch = pointcloud.shape[-1]
    nlayers = (len(sa_layers[0]), len(sa_layers[1]))
    wargs, wspecs = [], []
    for scale in sa_layers:
        for (w, bias) in scale:
            wargs += [w, bias]
            wspecs += [pl.BlockSpec(w.shape, lambda i: (0, 0)),
                       pl.BlockSpec(bias.shape, lambda i: (0, 0))]
    cout = sum(int(s[-1][0].shape[1]) for s in sa_layers)
    body = functools.partial(_group1_body, n, npoint, nsamples, nlayers)

    def wrapped(*refs):
        body(refs)

    return pl.pallas_call(
        wrapped,
        grid=(b,),
        in_specs=[
            pl.BlockSpec((1, n, ch), lambda i: (i, 0, 0)),
            pl.BlockSpec((1, npoint, 3), lambda i: (i, 0, 0)),
            pl.BlockSpec((1, npoint, nsamples[0]), lambda i: (i, 0, 0),
                         memory_space=pltpu.SMEM),
            pl.BlockSpec((1, npoint, nsamples[1]), lambda i: (i, 0, 0),
                         memory_space=pltpu.SMEM),
        ] + wspecs,
        out_specs=[pl.BlockSpec((1, npoint, cout), lambda i: (i, 0, 0))],
        out_shape=[jax.ShapeDtypeStruct((b, npoint, cout), F32)],
        scratch_shapes=[pltpu.VMEM((nsamples[0] * npoint, ch), F32),
                        pltpu.VMEM((nsamples[1] * npoint, ch), F32)],
        interpret=_INTERPRET,
    )(pointcloud, nx, i0, i1, *wargs)[0]


def _fps_batched(px, py, pz, npoint):
    b, p = px.shape
    niota = _iota((1, p), 1)
    citer = _iota((1, npoint), 1)

    def step(s, carry):
        dists, far, nxx, nxy, nxz = carry
        sel = (niota == far)
        cx = jnp.sum(jnp.where(sel, px, 0.0), axis=1, keepdims=True)
        cy = jnp.sum(jnp.where(sel, py, 0.0), axis=1, keepdims=True)
        cz = jnp.sum(jnp.where(sel, pz, 0.0), axis=1, keepdims=True)
        scol = (citer == s.astype(F32))
        nxx = jnp.where(scol, cx, nxx)
        nxy = jnp.where(scol, cy, nxy)
        nxz = jnp.where(scol, cz, nxz)
        dx = px - cx
        dy = py - cy
        dz = pz - cz
        d = dx * dx + dy * dy + dz * dz
        dists = jnp.minimum(dists, d)
        m = jnp.max(dists, axis=1, keepdims=True)
        far = jnp.min(jnp.where(dists == m, niota, BIG), axis=1, keepdims=True)
        return dists, far, nxx, nxy, nxz

    init = (jnp.full((b, p), 1e10, F32), jnp.zeros((b, 1), F32),
            jnp.zeros((b, npoint), F32), jnp.zeros((b, npoint), F32),
            jnp.zeros((b, npoint), F32))
    _, _, nxx, nxy, nxz = jax.lax.fori_loop(0, npoint, step, init)
    return nxx, nxy, nxz


def _fps23_body(np2, np3, xyz1T_ref, nx2_ref, nx3_ref):
    px = xyz1T_ref[:, 0, :]
    py = xyz1T_ref[:, 1, :]
    pz = xyz1T_ref[:, 2, :]
    n2x, n2y, n2z = _fps_batched(px, py, pz, np2)
    nx2_ref[:, 0, :] = n2x
    nx2_ref[:, 1, :] = n2y
    nx2_ref[:, 2, :] = n2z
    n3x, n3y, n3z = _fps_batched(n2x, n2y, n2z, np3)
    nx3_ref[:, 0, :] = n3x
    nx3_ref[:, 1, :] = n3y
    nx3_ref[:, 2, :] = n3z


def _fps23(xyz1T, b, np2, np3):
    p = xyz1T.shape[2]
    body = functools.partial(_fps23_body, np2, np3)
    nx2T, nx3T = pl.pallas_call(
        body,
        grid=(1,),
        in_specs=[pl.BlockSpec((b, 3, p), lambda i: (0, 0, 0))],
        out_specs=[
            pl.BlockSpec((b, 3, np2), lambda i: (0, 0, 0)),
            pl.BlockSpec((b, 3, np3), lambda i: (0, 0, 0)),
        ],
        out_shape=[
            jax.ShapeDtypeStruct((b, 3, np2), F32),
            jax.ShapeDtypeStruct((b, 3, np3), F32),
        ],
        interpret=_INTERPRET,
    )(xyz1T)
    return jnp.transpose(nx2T, (0, 2, 1)), jnp.transpose(nx3T, (0, 2, 1))


def _sqd_small(a, bpts):
    p = bpts.shape[0]
    ax, ay, az = a[:, 0:1], a[:, 1:2], a[:, 2:3]
    bx = bpts[:, 0:1].reshape(1, p)
    by = bpts[:, 1:2].reshape(1, p)
    bz = bpts[:, 2:3].reshape(1, p)
    aa = ax * ax + ay * ay + az * az
    bb = bx * bx + by * by + bz * bz
    ab = (_b16(ax) * _b16(bx) + _b16(ay) * _b16(by)) + _b16(az) * _b16(bz)
    return jnp.maximum(aa + bb - 2.0 * ab, 0.0)


def _ball_small(sqd, radius, k):
    s, p = sqd.shape
    niota = _iota((1, p), 1)
    pf = float(p)
    dsel = jnp.where(sqd < radius * radius, niota, pf)
    cols = []
    for _ in range(k):
        mk = jnp.min(dsel, axis=1, keepdims=True)
        cols.append(mk)
        dsel = jnp.where(dsel == mk, BIG, dsel)
    idx = jnp.concatenate(cols, axis=1)
    first = idx[:, 0:1]
    idx = jnp.where(idx >= pf, first, idx)
    idx = jnp.where(idx >= pf, 0.0, idx)
    return idx


def _gather_oh(idxcol, table):
    p = table.shape[0]
    oh = (_iota((1, p), 1) == idxcol).astype(F32)
    return jnp.dot(oh, table, preferred_element_type=F32)


def _sa_small(xyz, feats, nx, radii, nsamples, scales):
    p = xyz.shape[0]
    npoint = nx.shape[0]
    table = jnp.concatenate([xyz, feats], axis=1)
    sqd = _sqd_small(nx, xyz)
    outs = []
    for radius, k, layers in zip(radii, nsamples, scales):
        idx = _ball_small(sqd, radius, k)
        hs = []
        for j in range(k):
            rows = _gather_oh(idx[:, j:j + 1], table)
            hs.append(jnp.concatenate([rows[:, 0:3] - nx, rows[:, 3:]], axis=1))
        h = jnp.concatenate(hs, axis=0)
        h = _mlp_rows(h, layers)
        o = h[0:npoint]
        for j in range(1, k):
            o = jnp.maximum(o, h[j * npoint:(j + 1) * npoint])
        outs.append(o)
    return jnp.concatenate(outs, axis=1)


def _interp3(unknown, known, kn_f):
    u = unknown.shape[0]
    p = known.shape[0]
    sqd = _sqd_small(unknown, known)
    kiota = _iota((1, p), 1)
    d = sqd
    wmat = jnp.zeros((u, p), F32)
    for _ in range(3):
        m = jnp.min(d, axis=1, keepdims=True)
        idxk = jnp.min(jnp.where(d == m, kiota, BIG), axis=1, keepdims=True)
        oh = (kiota == idxk)
        w = 1.0 / (jnp.sqrt(jnp.maximum(m, 0.0)) + 1e-8)
        wmat = wmat + jnp.where(oh, w, 0.0)
        d = jnp.where(oh, BIG, d)
    wmat = wmat / jnp.sum(wmat, axis=1, keepdims=True)
    return jnp.dot(wmat, kn_f, preferred_element_type=F32)


def _mid_body(cfg2, cfg3, nl, args):
    xyz1_ref, f1_ref, nx2_ref, nx3_ref = args[:4]
    wrefs = args[4:4 + 2 * sum(nl)]
    out_ref = args[4 + 2 * sum(nl)]

    def take(off, count):
        return [(wrefs[off + 2 * j][...], wrefs[off + 2 * j + 1][...])
                for j in range(count)]

    o = 0
    sa2_l0 = take(o, nl[0]); o += 2 * nl[0]
    sa2_l1 = take(o, nl[1]); o += 2 * nl[1]
    sa3_l0 = take(o, nl[2]); o += 2 * nl[2]
    sa3_l1 = take(o, nl[3]); o += 2 * nl[3]
    fp2_l = take(o, nl[4]); o += 2 * nl[4]
    fp1_l = take(o, nl[5]); o += 2 * nl[5]

    xyz1 = xyz1_ref[0]
    f1 = f1_ref[0]
    xyz2 = nx2_ref[0]
    xyz3 = nx3_ref[0]
    _, radii2, ns2 = cfg2
    _, radii3, ns3 = cfg3
    f2 = _sa_small(xyz1, f1, xyz2, radii2, ns2, (sa2_l0, sa2_l1))
    f3 = _sa_small(xyz2, f2, xyz3, radii3, ns3, (sa3_l0, sa3_l1))
    h = jnp.concatenate([_interp3(xyz2, xyz3, f3), f2], axis=1)
    f2p = _mlp_rows(h, fp2_l)
    h = jnp.concatenate([_interp3(xyz1, xyz2, f2p), f1], axis=1)
    out_ref[0] = _mlp_rows(h, fp1_l)


def _mid(nx1, f1, nx2, nx3, wlists, cfg2, cfg3, b):
    npoint = nx1.shape[1]
    np2 = nx2.shape[1]
    np3 = nx3.shape[1]
    c1 = f1.shape[-1]
    nl = tuple(len(l) for l in wlists)
    wargs, wspecs = [], []
    for lst in wlists:
        for (w, bias) in lst:
            wargs += [w, bias]
            wspecs += [pl.BlockSpec(w.shape, lambda i: (0, 0)),
                       pl.BlockSpec(bias.shape, lambda i: (0, 0))]
    cout = int(wlists[-1][-1][0].shape[1])
    body = functools.partial(_mid_body, cfg2, cfg3, nl)

    def wrapped(*refs):
        body(refs)

    return pl.pallas_call(
        wrapped,
        grid=(b,),
        in_specs=[
            pl.BlockSpec((1, npoint, 3), lambda i: (i, 0, 0)),
            pl.BlockSpec((1, npoint, c1), lambda i: (i, 0, 0)),
            pl.BlockSpec((1, np2, 3), lambda i: (i, 0, 0)),
            pl.BlockSpec((1, np3, 3), lambda i: (i, 0, 0)),
        ] + wspecs,
        out_specs=[pl.BlockSpec((1, npoint, cout), lambda i: (i, 0, 0))],
        out_shape=[jax.ShapeDtypeStruct((b, npoint, cout), F32)],
        interpret=_INTERPRET,
    )(nx1, f1, nx2, nx3, *wargs)[0]


def _head_body(nl, args):
    pc_ref, nx_ref, f1p_ref = args[:3]
    wrefs = args[3:3 + 2 * nl]
    out_ref = args[3 + 2 * nl]
    layers = [(wrefs[2 * j][...], wrefs[2 * j + 1][...]) for j in range(nl)]
    pc = pc_ref[0]
    xt = pc[:, 0:3]
    ft = pc[:, 3:]
    interp = _interp3(xt, nx_ref[0], f1p_ref[0])
    h = jnp.concatenate([interp, ft], axis=1)
    nrelu = nl - 1
    for j in range(nrelu):
        w, bias = layers[j]
        h = jnp.maximum(jnp.dot(h, w, preferred_element_type=F32) + bias, 0.0)
    w, bias = layers[nrelu]
    out_ref[0] = jnp.dot(h, w, preferred_element_type=F32) + bias


def _head(pointcloud, nx1, f1p, layers, b, n, tile):
    ch = pointcloud.shape[-1]
    npoint = nx1.shape[1]
    c1 = f1p.shape[-1]
    wargs, wspecs = [], []
    for (w, bias) in layers:
        wargs += [w, bias]
        wspecs += [pl.BlockSpec(w.shape, lambda i, t: (0, 0)),
                   pl.BlockSpec(bias.shape, lambda i, t: (0, 0))]
    ncls = int(layers[-1][0].shape[1])
    body = functools.partial(_head_body, len(layers))

    def wrapped(*refs):
        body(refs)

    return pl.pallas_call(
        wrapped,
        grid=(b, n // tile),
        in_specs=[
            pl.BlockSpec((1, tile, ch), lambda i, t: (i, t, 0)),
            pl.BlockSpec((1, npoint, 3), lambda i, t: (i, 0, 0)),
            pl.BlockSpec((1, npoint, c1), lambda i, t: (i, 0, 0)),
        ] + wspecs,
        out_specs=[pl.BlockSpec((1, tile, ncls), lambda i, t: (i, t, 0))],
        out_shape=[jax.ShapeDtypeStruct((b, n, ncls), F32)],
        interpret=_INTERPRET,
    )(pointcloud, nx1, f1p, *wargs)[0]


def _fold(layers):
    out = []
    for (w, g, bias) in layers:
        out.append(((w * (g * _BN_INV)[:, None]).T, bias.reshape(1, -1)))
    return out


def kernel(pointcloud, params):
    b, n, ch = pointcloud.shape
    xyzT = jnp.transpose(pointcloud[..., :3], (0, 2, 1))

    sa_cfg = [(128, (1.0, 3.0), (2, 8)),
              (128, (2.0, 4.0), (2, 8)),
              (64, (3.0, 6.0), (4, 8))]

    np1, radii1, ns1 = sa_cfg[0]
    nx1T = _fps1(xyzT, n, b, np1)
    nx1 = jnp.transpose(nx1T, (0, 2, 1))
    i0, i1 = _bq1(xyzT, nx1T, n, b, np1, radii1, ns1)

    sa1_layers = [_fold(s) for s in params['sa'][0]]
    f1 = _group1(pointcloud, nx1, i0, i1, sa1_layers, n, b, np1, ns1)

    nx2, nx3 = _fps23(nx1T, b, sa_cfg[1][0], sa_cfg[2][0])
    wlists = ([_fold(s) for s in params['sa'][1]]
              + [_fold(s) for s in params['sa'][2]]
              + [_fold(params['fp'][2]), _fold(params['fp'][1])])
    f1p = _mid(nx1, f1, nx2, nx3, wlists, sa_cfg[1], sa_cfg[2], b)

    fc = params['fc']
    head_layers = _fold(params['fp'][0]) + [
        ((fc['W1'] * (fc['g1'] * _BN_INV)[:, None]).T, fc['b1'].reshape(1, -1)),
        (fc['W2'].T, fc['b2'].reshape(1, -1)),
        (fc['W3'].T, fc['b3'].reshape(1, -1)),
    ]
    return _head(pointcloud, nx1, f1p, head_layers, b, n, 2048)


# speedup vs baseline: 28.1820x; 1.0813x over previous
import functools
import numpy as np
import jax
import jax.numpy as jnp
from jax.experimental import pallas as pl
from jax.experimental.pallas import tpu as pltpu

_INTERPRET = False

_BN_INV = 1.0 / np.sqrt(1.0 + 1e-5)
F32 = jnp.float32
BIG = 3.0e7


def _iota(shape, dim):
    return jax.lax.broadcasted_iota(jnp.int32, shape, dim).astype(F32)


def _b16(x):
    return x.astype(jnp.bfloat16).astype(F32)


def _fps1_body(npoint, xyzT_ref, nx_ref):
    px = xyzT_ref[:, 0, :]
    py = xyzT_ref[:, 1, :]
    pz = xyzT_ref[:, 2, :]
    nxx, nxy, nxz = _fps_batched(px, py, pz, npoint)
    nx_ref[:, 0, :] = nxx
    nx_ref[:, 1, :] = nxy
    nx_ref[:, 2, :] = nxz


def _fps1(xyzT, n, b, npoint):
    body = functools.partial(_fps1_body, npoint)
    return pl.pallas_call(
        body,
        grid=(1,),
        in_specs=[pl.BlockSpec((b, 3, n), lambda i: (0, 0, 0))],
        out_specs=[pl.BlockSpec((b, 3, npoint), lambda i: (0, 0, 0))],
        out_shape=[jax.ShapeDtypeStruct((b, 3, npoint), F32)],
        interpret=_INTERPRET,
    )(xyzT)[0]


def _bq1_body(n, npoint, radii, nsamples, xyzT_ref, nxT_ref, i0_ref, i1_ref):
    px = xyzT_ref[0, 0:1, :]
    py = xyzT_ref[0, 1:2, :]
    pz = xyzT_ref[0, 2:3, :]
    nxx = nxT_ref[0, 0:1, :].reshape(npoint, 1)
    nxy = nxT_ref[0, 1:2, :].reshape(npoint, 1)
    nxz = nxT_ref[0, 2:3, :].reshape(npoint, 1)
    aa = nxx * nxx + nxy * nxy + nxz * nxz
    bb = px * px + py * py + pz * pz
    ab = (_b16(nxx) * _b16(px) + _b16(nxy) * _b16(py)) + _b16(nxz) * _b16(pz)
    sqd = jnp.maximum(aa + bb - 2.0 * ab, 0.0)
    niota = _iota((1, n), 1)
    nf = float(n)
    for (radius, k, out_ref) in ((radii[0], nsamples[0], i0_ref),
                                 (radii[1], nsamples[1], i1_ref)):
        dsel = jnp.where(sqd < radius * radius, niota, nf)
        cols = [jnp.min(dsel, axis=1, keepdims=True)]
        for _ in range(k - 1):
            mk = jnp.min(jnp.where(dsel > cols[-1], dsel, BIG),
                         axis=1, keepdims=True)
            cols.append(mk)
        idx = jnp.concatenate(cols, axis=1)
        first = idx[:, 0:1]
        idx = jnp.where(idx >= nf, first, idx)
        idx = jnp.where(idx >= nf, 0.0, idx)
        out_ref[0] = idx.astype(jnp.int32)


def _bq1(xyzT, nxT, n, b, npoint, radii, nsamples):
    body = functools.partial(_bq1_body, n, npoint, radii, nsamples)
    return pl.pallas_call(
        body,
        grid=(b,),
        in_specs=[
            pl.BlockSpec((1, 3, n), lambda i: (i, 0, 0)),
            pl.BlockSpec((1, 3, npoint), lambda i: (i, 0, 0)),
        ],
        out_specs=[
            pl.BlockSpec((1, npoint, nsamples[0]), lambda i: (i, 0, 0)),
            pl.BlockSpec((1, npoint, nsamples[1]), lambda i: (i, 0, 0)),
        ],
        out_shape=[
            jax.ShapeDtypeStruct((b, npoint, nsamples[0]), jnp.int32),
            jax.ShapeDtypeStruct((b, npoint, nsamples[1]), jnp.int32),
        ],
        interpret=_INTERPRET,
    )(xyzT, nxT)


def _mlp_rows(h, layers):
    for (w, bias) in layers:
        h = jnp.maximum(jnp.dot(h, w, preferred_element_type=F32) + bias, 0.0)
    return h


def _group1_body(n, npoint, nsamples, nlayers, args):
    pc_ref, nx_ref, i0_ref, i1_ref = args[:4]
    wrefs = args[4:4 + 2 * nlayers[0] + 2 * nlayers[1]]
    f_ref = args[4 + 2 * nlayers[0] + 2 * nlayers[1]]
    g0_ref, g1_ref = args[-2:]
    k0, k1 = nsamples

    def gather(s, _):
        for k in range(k0):
            i = i0_ref[0, s, k]
            g0_ref[pl.ds(k * npoint + s, 1), :] = pc_ref[0, pl.ds(i, 1), :]
        for k in range(k1):
            i = i1_ref[0, s, k]
            g1_ref[pl.ds(k * npoint + s, 1), :] = pc_ref[0, pl.ds(i, 1), :]
        return 0

    jax.lax.fori_loop(0, npoint, gather, 0)
    nx = nx_ref[0]
    outs = []
    woff = 0
    for (k, g_ref, nl) in ((k0, g0_ref, nlayers[0]), (k1, g1_ref, nlayers[1])):
        layers = [(wrefs[woff + 2 * j][...], wrefs[woff + 2 * j + 1][...])
                  for j in range(nl)]
        woff += 2 * nl
        g = g_ref[...]
        cent = jnp.concatenate([nx] * k, axis=0)
        h = jnp.concatenate([g[:, 0:3] - cent, g[:, 3:]], axis=1)
        h = _mlp_rows(h, layers)
        o = h[0:npoint]
        for j in range(1, k):
            o = jnp.maximum(o, h[j * npoint:(j + 1) * npoint])
        outs.append(o)
    f_ref[0] = jnp.concatenate(outs, axis=1)


def _group1(pointcloud, nx, i0, i1, sa_layers, n, b, npoint, nsamples):
    ch = pointcloud.shape[-1]
    nlayers = (len(sa_layers[0]), len(sa_layers[1]))
    wargs, wspecs = [], []
    for scale in sa_layers:
        for (w, bias) in scale:
            wargs += [w, bias]
            wspecs += [pl.BlockSpec(w.shape, lambda i: (0, 0)),
                       pl.BlockSpec(bias.shape, lambda i: (0, 0))]
    cout = sum(int(s[-1][0].shape[1]) for s in sa_layers)
    body = functools.partial(_group1_body, n, npoint, nsamples, nlayers)

    def wrapped(*refs):
        body(refs)

    return pl.pallas_call(
        wrapped,
        grid=(b,),
        in_specs=[
            pl.BlockSpec((1, n, ch), lambda i: (i, 0, 0)),
            pl.BlockSpec((1, npoint, 3), lambda i: (i, 0, 0)),
            pl.BlockSpec((1, npoint, nsamples[0]), lambda i: (i, 0, 0),
                         memory_space=pltpu.SMEM),
            pl.BlockSpec((1, npoint, nsamples[1]), lambda i: (i, 0, 0),
                         memory_space=pltpu.SMEM),
        ] + wspecs,
        out_specs=[pl.BlockSpec((1, npoint, cout), lambda i: (i, 0, 0))],
        out_shape=[jax.ShapeDtypeStruct((b, npoint, cout), F32)],
        scratch_shapes=[pltpu.VMEM((nsamples[0] * npoint, ch), F32),
                        pltpu.VMEM((nsamples[1] * npoint, ch), F32)],
        interpret=_INTERPRET,
    )(pointcloud, nx, i0, i1, *wargs)[0]


def _fps_batched(px, py, pz, npoint):
    b, p = px.shape
    niota = _iota((1, p), 1)
    citer = _iota((1, npoint), 1)

    def step(s, carry):
        dists, far, nxx, nxy, nxz = carry
        sel = (niota == far)
        cx = jnp.sum(jnp.where(sel, px, 0.0), axis=1, keepdims=True)
        cy = jnp.sum(jnp.where(sel, py, 0.0), axis=1, keepdims=True)
        cz = jnp.sum(jnp.where(sel, pz, 0.0), axis=1, keepdims=True)
        scol = (citer == s.astype(F32))
        nxx = jnp.where(scol, cx, nxx)
        nxy = jnp.where(scol, cy, nxy)
        nxz = jnp.where(scol, cz, nxz)
        dx = px - cx
        dy = py - cy
        dz = pz - cz
        d = dx * dx + dy * dy + dz * dz
        dists = jnp.minimum(dists, d)
        m = jnp.max(dists, axis=1, keepdims=True)
        far = jnp.min(jnp.where(dists == m, niota, BIG), axis=1, keepdims=True)
        return dists, far, nxx, nxy, nxz

    init = (jnp.full((b, p), 1e10, F32), jnp.zeros((b, 1), F32),
            jnp.zeros((b, npoint), F32), jnp.zeros((b, npoint), F32),
            jnp.zeros((b, npoint), F32))
    _, _, nxx, nxy, nxz = jax.lax.fori_loop(0, npoint, step, init)
    return nxx, nxy, nxz


def _fps23_body(np2, np3, xyz1T_ref, nx2_ref, nx3_ref):
    px = xyz1T_ref[:, 0, :]
    py = xyz1T_ref[:, 1, :]
    pz = xyz1T_ref[:, 2, :]
    n2x, n2y, n2z = _fps_batched(px, py, pz, np2)
    nx2_ref[:, 0, :] = n2x
    nx2_ref[:, 1, :] = n2y
    nx2_ref[:, 2, :] = n2z
    n3x, n3y, n3z = _fps_batched(n2x, n2y, n2z, np3)
    nx3_ref[:, 0, :] = n3x
    nx3_ref[:, 1, :] = n3y
    nx3_ref[:, 2, :] = n3z


def _fps23(xyz1T, b, np2, np3):
    p = xyz1T.shape[2]
    body = functools.partial(_fps23_body, np2, np3)
    nx2T, nx3T = pl.pallas_call(
        body,
        grid=(1,),
        in_specs=[pl.BlockSpec((b, 3, p), lambda i: (0, 0, 0))],
        out_specs=[
            pl.BlockSpec((b, 3, np2), lambda i: (0, 0, 0)),
            pl.BlockSpec((b, 3, np3), lambda i: (0, 0, 0)),
        ],
        out_shape=[
            jax.ShapeDtypeStruct((b, 3, np2), F32),
            jax.ShapeDtypeStruct((b, 3, np3), F32),
        ],
        interpret=_INTERPRET,
    )(xyz1T)
    return jnp.transpose(nx2T, (0, 2, 1)), jnp.transpose(nx3T, (0, 2, 1))


def _sqd_small(a, bpts):
    p = bpts.shape[0]
    ax, ay, az = a[:, 0:1], a[:, 1:2], a[:, 2:3]
    bx = bpts[:, 0:1].reshape(1, p)
    by = bpts[:, 1:2].reshape(1, p)
    bz = bpts[:, 2:3].reshape(1, p)
    aa = ax * ax + ay * ay + az * az
    bb = bx * bx + by * by + bz * bz
    ab = (_b16(ax) * _b16(bx) + _b16(ay) * _b16(by)) + _b16(az) * _b16(bz)
    return jnp.maximum(aa + bb - 2.0 * ab, 0.0)


def _ball_small(sqd, radius, k):
    s, p = sqd.shape
    niota = _iota((1, p), 1)
    pf = float(p)
    dsel = jnp.where(sqd < radius * radius, niota, pf)
    cols = [jnp.min(dsel, axis=1, keepdims=True)]
    for _ in range(k - 1):
        mk = jnp.min(jnp.where(dsel > cols[-1], dsel, BIG),
                     axis=1, keepdims=True)
        cols.append(mk)
    idx = jnp.concatenate(cols, axis=1)
    first = idx[:, 0:1]
    idx = jnp.where(idx >= pf, first, idx)
    idx = jnp.where(idx >= pf, 0.0, idx)
    return idx


def _gather_oh(idxcol, table):
    p = table.shape[0]
    oh = (_iota((1, p), 1) == idxcol).astype(F32)
    return jnp.dot(oh, table, preferred_element_type=F32)


def _sa_small(xyz, feats, nx, radii, nsamples, scales):
    p = xyz.shape[0]
    npoint = nx.shape[0]
    table = jnp.concatenate([xyz, feats], axis=1)
    sqd = _sqd_small(nx, xyz)
    outs = []
    for radius, k, layers in zip(radii, nsamples, scales):
        idx = _ball_small(sqd, radius, k)
        hs = []
        for j in range(k):
            rows = _gather_oh(idx[:, j:j + 1], table)
            hs.append(jnp.concatenate([rows[:, 0:3] - nx, rows[:, 3:]], axis=1))
        h = jnp.concatenate(hs, axis=0)
        h = _mlp_rows(h, layers)
        o = h[0:npoint]
        for j in range(1, k):
            o = jnp.maximum(o, h[j * npoint:(j + 1) * npoint])
        outs.append(o)
    return jnp.concatenate(outs, axis=1)


def _interp3(unknown, known, kn_f):
    sqd = _sqd_small(unknown, known)
    m1 = jnp.min(sqd, axis=1, keepdims=True)
    m2 = jnp.min(jnp.where(sqd > m1, sqd, BIG), axis=1, keepdims=True)
    m3 = jnp.min(jnp.where(sqd > m2, sqd, BIG), axis=1, keepdims=True)
    w = jnp.where(sqd <= m3, 1.0 / (jnp.sqrt(sqd) + 1e-8), 0.0)
    w = w / jnp.sum(w, axis=1, keepdims=True)
    return jnp.dot(w, kn_f, preferred_element_type=F32)


def _mid_body(cfg2, cfg3, nl, args):
    xyz1_ref, f1_ref, nx2_ref, nx3_ref = args[:4]
    wrefs = args[4:4 + 2 * sum(nl)]
    out_ref = args[4 + 2 * sum(nl)]

    def take(off, count):
        return [(wrefs[off + 2 * j][...], wrefs[off + 2 * j + 1][...])
                for j in range(count)]

    o = 0
    sa2_l0 = take(o, nl[0]); o += 2 * nl[0]
    sa2_l1 = take(o, nl[1]); o += 2 * nl[1]
    sa3_l0 = take(o, nl[2]); o += 2 * nl[2]
    sa3_l1 = take(o, nl[3]); o += 2 * nl[3]
    fp2_l = take(o, nl[4]); o += 2 * nl[4]
    fp1_l = take(o, nl[5]); o += 2 * nl[5]

    xyz1 = xyz1_ref[0]
    f1 = f1_ref[0]
    xyz2 = nx2_ref[0]
    xyz3 = nx3_ref[0]
    _, radii2, ns2 = cfg2
    _, radii3, ns3 = cfg3
    f2 = _sa_small(xyz1, f1, xyz2, radii2, ns2, (sa2_l0, sa2_l1))
    f3 = _sa_small(xyz2, f2, xyz3, radii3, ns3, (sa3_l0, sa3_l1))
    h = jnp.concatenate([_interp3(xyz2, xyz3, f3), f2], axis=1)
    f2p = _mlp_rows(h, fp2_l)
    h = jnp.concatenate([_interp3(xyz1, xyz2, f2p), f1], axis=1)
    out_ref[0] = _mlp_rows(h, fp1_l)


def _mid(nx1, f1, nx2, nx3, wlists, cfg2, cfg3, b):
    npoint = nx1.shape[1]
    np2 = nx2.shape[1]
    np3 = nx3.shape[1]
    c1 = f1.shape[-1]
    nl = tuple(len(l) for l in wlists)
    wargs, wspecs = [], []
    for lst in wlists:
        for (w, bias) in lst:
            wargs += [w, bias]
            wspecs += [pl.BlockSpec(w.shape, lambda i: (0, 0)),
                       pl.BlockSpec(bias.shape, lambda i: (0, 0))]
    cout = int(wlists[-1][-1][0].shape[1])
    body = functools.partial(_mid_body, cfg2, cfg3, nl)

    def wrapped(*refs):
        body(refs)

    return pl.pallas_call(
        wrapped,
        grid=(b,),
        in_specs=[
            pl.BlockSpec((1, npoint, 3), lambda i: (i, 0, 0)),
            pl.BlockSpec((1, npoint, c1), lambda i: (i, 0, 0)),
            pl.BlockSpec((1, np2, 3), lambda i: (i, 0, 0)),
            pl.BlockSpec((1, np3, 3), lambda i: (i, 0, 0)),
        ] + wspecs,
        out_specs=[pl.BlockSpec((1, npoint, cout), lambda i: (i, 0, 0))],
        out_shape=[jax.ShapeDtypeStruct((b, npoint, cout), F32)],
        interpret=_INTERPRET,
    )(nx1, f1, nx2, nx3, *wargs)[0]


def _head_body(nl, args):
    pc_ref, nx_ref, f1p_ref = args[:3]
    wrefs = args[3:3 + 2 * nl]
    out_ref = args[3 + 2 * nl]
    layers = [(wrefs[2 * j][...], wrefs[2 * j + 1][...]) for j in range(nl)]
    pc = pc_ref[0]
    xt = pc[:, 0:3]
    ft = pc[:, 3:]
    interp = _interp3(xt, nx_ref[0], f1p_ref[0])
    h = jnp.concatenate([interp, ft], axis=1)
    nrelu = nl - 1
    for j in range(nrelu):
        w, bias = layers[j]
        h = jnp.maximum(jnp.dot(h, w, preferred_element_type=F32) + bias, 0.0)
    w, bias = layers[nrelu]
    out_ref[0] = jnp.dot(h, w, preferred_element_type=F32) + bias


def _head(pointcloud, nx1, f1p, layers, b, n, tile):
    ch = pointcloud.shape[-1]
    npoint = nx1.shape[1]
    c1 = f1p.shape[-1]
    wargs, wspecs = [], []
    for (w, bias) in layers:
        wargs += [w, bias]
        wspecs += [pl.BlockSpec(w.shape, lambda i, t: (0, 0)),
                   pl.BlockSpec(bias.shape, lambda i, t: (0, 0))]
    ncls = int(layers[-1][0].shape[1])
    body = functools.partial(_head_body, len(layers))

    def wrapped(*refs):
        body(refs)

    return pl.pallas_call(
        wrapped,
        grid=(b, n // tile),
        in_specs=[
            pl.BlockSpec((1, tile, ch), lambda i, t: (i, t, 0)),
            pl.BlockSpec((1, npoint, 3), lambda i, t: (i, 0, 0)),
            pl.BlockSpec((1, npoint, c1), lambda i, t: (i, 0, 0)),
        ] + wspecs,
        out_specs=[pl.BlockSpec((1, tile, ncls), lambda i, t: (i, t, 0))],
        out_shape=[jax.ShapeDtypeStruct((b, n, ncls), F32)],
        interpret=_INTERPRET,
    )(pointcloud, nx1, f1p, *wargs)[0]


def _fold(layers):
    out = []
    for (w, g, bias) in layers:
        out.append(((w * (g * _BN_INV)[:, None]).T, bias.reshape(1, -1)))
    return out


def kernel(pointcloud, params):
    b, n, ch = pointcloud.shape
    xyzT = jnp.transpose(pointcloud[..., :3], (0, 2, 1))

    sa_cfg = [(128, (1.0, 3.0), (2, 8)),
              (128, (2.0, 4.0), (2, 8)),
              (64, (3.0, 6.0), (4, 8))]

    np1, radii1, ns1 = sa_cfg[0]
    nx1T = _fps1(xyzT, n, b, np1)
    nx1 = jnp.transpose(nx1T, (0, 2, 1))
    i0, i1 = _bq1(xyzT, nx1T, n, b, np1, radii1, ns1)

    sa1_layers = [_fold(s) for s in params['sa'][0]]
    f1 = _group1(pointcloud, nx1, i0, i1, sa1_layers, n, b, np1, ns1)

    nx2, nx3 = _fps23(nx1T, b, sa_cfg[1][0], sa_cfg[2][0])
    wlists = ([_fold(s) for s in params['sa'][1]]
              + [_fold(s) for s in params['sa'][2]]
              + [_fold(params['fp'][2]), _fold(params['fp'][1])])
    f1p = _mid(nx1, f1, nx2, nx3, wlists, sa_cfg[1], sa_cfg[2], b)

    fc = params['fc']
    head_layers = _fold(params['fp'][0]) + [
        ((fc['W1'] * (fc['g1'] * _BN_INV)[:, None]).T, fc['b1'].reshape(1, -1)),
        (fc['W2'].T, fc['b2'].reshape(1, -1)),
        (fc['W3'].T, fc['b3'].reshape(1, -1)),
    ]
    return _head(pointcloud, nx1, f1p, head_layers, b, n, 4096)


# speedup vs baseline: 36.6083x; 1.2990x over previous
import functools
import numpy as np
import jax
import jax.numpy as jnp
from jax.experimental import pallas as pl
from jax.experimental.pallas import tpu as pltpu

_INTERPRET = False

_BN_INV = 1.0 / np.sqrt(1.0 + 1e-5)
F32 = jnp.float32
BIG = 3.0e7


def _iota(shape, dim):
    return jax.lax.broadcasted_iota(jnp.int32, shape, dim).astype(F32)


def _b16(x):
    return x.astype(jnp.bfloat16).astype(F32)


def _fps1_body(npoint, xyzT_ref, nx_ref):
    px = xyzT_ref[:, 0, :]
    py = xyzT_ref[:, 1, :]
    pz = xyzT_ref[:, 2, :]
    nxx, nxy, nxz = _fps_batched(px, py, pz, npoint)
    nx_ref[:, 0, :] = nxx
    nx_ref[:, 1, :] = nxy
    nx_ref[:, 2, :] = nxz


def _fps1(xyzT, n, b, npoint):
    body = functools.partial(_fps1_body, npoint)
    return pl.pallas_call(
        body,
        grid=(1,),
        in_specs=[pl.BlockSpec((b, 3, n), lambda i: (0, 0, 0))],
        out_specs=[pl.BlockSpec((b, 3, npoint), lambda i: (0, 0, 0))],
        out_shape=[jax.ShapeDtypeStruct((b, 3, npoint), F32)],
        interpret=_INTERPRET,
    )(xyzT)[0]


def _bq1_body(n, npoint, radii, nsamples, xyzT_ref, nxT_ref, nx_ref,
              i0_ref, i1_ref):
    px = xyzT_ref[0, 0:1, :]
    py = xyzT_ref[0, 1:2, :]
    pz = xyzT_ref[0, 2:3, :]
    nxx = nxT_ref[0, 0:1, :].reshape(npoint, 1)
    nxy = nxT_ref[0, 1:2, :].reshape(npoint, 1)
    nxz = nxT_ref[0, 2:3, :].reshape(npoint, 1)
    aa = nxx * nxx + nxy * nxy + nxz * nxz
    bb = px * px + py * py + pz * pz
    ab = jnp.dot(nx_ref[0].astype(jnp.bfloat16),
                 xyzT_ref[0].astype(jnp.bfloat16),
                 preferred_element_type=F32)
    sqd = jnp.maximum(aa + bb - 2.0 * ab, 0.0)
    niota = _iota((1, n), 1)
    nf = float(n)
    for (radius, k, out_ref) in ((radii[0], nsamples[0], i0_ref),
                                 (radii[1], nsamples[1], i1_ref)):
        dsel = jnp.where(sqd < radius * radius, niota, nf)
        cols = [jnp.min(dsel, axis=1, keepdims=True)]
        for _ in range(k - 1):
            mk = jnp.min(jnp.where(dsel > cols[-1], dsel, BIG),
                         axis=1, keepdims=True)
            cols.append(mk)
        idx = jnp.concatenate(cols, axis=1)
        first = idx[:, 0:1]
        idx = jnp.where(idx >= nf, first, idx)
        idx = jnp.where(idx >= nf, 0.0, idx)
        out_ref[0] = idx.astype(jnp.int32)


def _bq1(xyzT, nxT, nx, n, b, npoint, radii, nsamples):
    body = functools.partial(_bq1_body, n, npoint, radii, nsamples)
    return pl.pallas_call(
        body,
        grid=(b,),
        in_specs=[
            pl.BlockSpec((1, 3, n), lambda i: (i, 0, 0)),
            pl.BlockSpec((1, 3, npoint), lambda i: (i, 0, 0)),
            pl.BlockSpec((1, npoint, 3), lambda i: (i, 0, 0)),
        ],
        out_specs=[
            pl.BlockSpec((1, npoint, nsamples[0]), lambda i: (i, 0, 0)),
            pl.BlockSpec((1, npoint, nsamples[1]), lambda i: (i, 0, 0)),
        ],
        out_shape=[
            jax.ShapeDtypeStruct((b, npoint, nsamples[0]), jnp.int32),
            jax.ShapeDtypeStruct((b, npoint, nsamples[1]), jnp.int32),
        ],
        interpret=_INTERPRET,
    )(xyzT, nxT, nx)


def _mlp_rows(h, layers):
    for (w, bias) in layers:
        h = jnp.maximum(jnp.dot(h, w, preferred_element_type=F32) + bias, 0.0)
    return h


def _group1_body(n, npoint, nsamples, nlayers, args):
    pc_ref, nx_ref, i0_ref, i1_ref = args[:4]
    wrefs = args[4:4 + 2 * nlayers[0] + 2 * nlayers[1]]
    f_ref = args[4 + 2 * nlayers[0] + 2 * nlayers[1]]
    g0_ref, g1_ref = args[-2:]
    k0, k1 = nsamples

    def gather(s, _):
        for k in range(k0):
            i = i0_ref[0, s, k]
            g0_ref[pl.ds(k * npoint + s, 1), :] = pc_ref[0, pl.ds(i, 1), :]
        for k in range(k1):
            i = i1_ref[0, s, k]
            g1_ref[pl.ds(k * npoint + s, 1), :] = pc_ref[0, pl.ds(i, 1), :]
        return 0

    jax.lax.fori_loop(0, npoint, gather, 0)
    nx = nx_ref[0]
    outs = []
    woff = 0
    for (k, g_ref, nl) in ((k0, g0_ref, nlayers[0]), (k1, g1_ref, nlayers[1])):
        layers = [(wrefs[woff + 2 * j][...], wrefs[woff + 2 * j + 1][...])
                  for j in range(nl)]
        woff += 2 * nl
        g = g_ref[...]
        cent = jnp.concatenate([nx] * k, axis=0)
        h = jnp.concatenate([g[:, 0:3] - cent, g[:, 3:]], axis=1)
        h = _mlp_rows(h, layers)
        o = h[0:npoint]
        for j in range(1, k):
            o = jnp.maximum(o, h[j * npoint:(j + 1) * npoint])
        outs.append(o)
    f_ref[0] = jnp.concatenate(outs, axis=1)


def _group1(pointcloud, nx, i0, i1, sa_layers, n, b, npoint, nsamples):
    ch = pointcloud.shape[-1]
    nlayers = (len(sa_layers[0]), len(sa_layers[1]))
    wargs, wspecs = [], []
    for scale in sa_layers:
        for (w, bias) in scale:
            wargs += [w, bias]
            wspecs += [pl.BlockSpec(w.shape, lambda i: (0, 0)),
                       pl.BlockSpec(bias.shape, lambda i: (0, 0))]
    cout = sum(int(s[-1][0].shape[1]) for s in sa_layers)
    body = functools.partial(_group1_body, n, npoint, nsamples, nlayers)

    def wrapped(*refs):
        body(refs)

    return pl.pallas_call(
        wrapped,
        grid=(b,),
        in_specs=[
            pl.BlockSpec((1, n, ch), lambda i: (i, 0, 0)),
            pl.BlockSpec((1, npoint, 3), lambda i: (i, 0, 0)),
            pl.BlockSpec((1, npoint, nsamples[0]), lambda i: (i, 0, 0),
                         memory_space=pltpu.SMEM),
            pl.BlockSpec((1, npoint, nsamples[1]), lambda i: (i, 0, 0),
                         memory_space=pltpu.SMEM),
        ] + wspecs,
        out_specs=[pl.BlockSpec((1, npoint, cout), lambda i: (i, 0, 0))],
        out_shape=[jax.ShapeDtypeStruct((b, npoint, cout), F32)],
        scratch_shapes=[pltpu.VMEM((nsamples[0] * npoint, ch), F32),
                        pltpu.VMEM((nsamples[1] * npoint, ch), F32)],
        interpret=_INTERPRET,
    )(pointcloud, nx, i0, i1, *wargs)[0]


def _fps_batched(px, py, pz, npoint):
    b, p = px.shape
    niota = _iota((1, p), 1)
    citer = _iota((1, npoint), 1)

    def step(s, carry):
        dists, far, nxx, nxy, nxz = carry
        sel = (niota == far)
        cx = jnp.sum(jnp.where(sel, px, 0.0), axis=1, keepdims=True)
        cy = jnp.sum(jnp.where(sel, py, 0.0), axis=1, keepdims=True)
        cz = jnp.sum(jnp.where(sel, pz, 0.0), axis=1, keepdims=True)
        scol = (citer == s.astype(F32))
        nxx = jnp.where(scol, cx, nxx)
        nxy = jnp.where(scol, cy, nxy)
        nxz = jnp.where(scol, cz, nxz)
        dx = px - cx
        dy = py - cy
        dz = pz - cz
        d = dx * dx + dy * dy + dz * dz
        dists = jnp.minimum(dists, d)
        m = jnp.max(dists, axis=1, keepdims=True)
        far = jnp.min(jnp.where(dists == m, niota, BIG), axis=1, keepdims=True)
        return dists, far, nxx, nxy, nxz

    init = (jnp.full((b, p), 1e10, F32), jnp.zeros((b, 1), F32),
            jnp.zeros((b, npoint), F32), jnp.zeros((b, npoint), F32),
            jnp.zeros((b, npoint), F32))
    _, _, nxx, nxy, nxz = jax.lax.fori_loop(0, npoint, step, init)
    return nxx, nxy, nxz


def _fps23_body(np2, np3, xyz1T_ref, nx2_ref, nx3_ref):
    px = xyz1T_ref[:, 0, :]
    py = xyz1T_ref[:, 1, :]
    pz = xyz1T_ref[:, 2, :]
    n2x, n2y, n2z = _fps_batched(px, py, pz, np2)
    nx2_ref[:, 0, :] = n2x
    nx2_ref[:, 1, :] = n2y
    nx2_ref[:, 2, :] = n2z
    n3x, n3y, n3z = _fps_batched(n2x, n2y, n2z, np3)
    nx3_ref[:, 0, :] = n3x
    nx3_ref[:, 1, :] = n3y
    nx3_ref[:, 2, :] = n3z


def _fps23(xyz1T, b, np2, np3):
    p = xyz1T.shape[2]
    body = functools.partial(_fps23_body, np2, np3)
    nx2T, nx3T = pl.pallas_call(
        body,
        grid=(1,),
        in_specs=[pl.BlockSpec((b, 3, p), lambda i: (0, 0, 0))],
        out_specs=[
            pl.BlockSpec((b, 3, np2), lambda i: (0, 0, 0)),
            pl.BlockSpec((b, 3, np3), lambda i: (0, 0, 0)),
        ],
        out_shape=[
            jax.ShapeDtypeStruct((b, 3, np2), F32),
            jax.ShapeDtypeStruct((b, 3, np3), F32),
        ],
        interpret=_INTERPRET,
    )(xyz1T)
    return jnp.transpose(nx2T, (0, 2, 1)), jnp.transpose(nx3T, (0, 2, 1))


def _sqd_small(a, bpts):
    p = bpts.shape[0]
    ax, ay, az = a[:, 0:1], a[:, 1:2], a[:, 2:3]
    bx = bpts[:, 0:1].reshape(1, p)
    by = bpts[:, 1:2].reshape(1, p)
    bz = bpts[:, 2:3].reshape(1, p)
    aa = ax * ax + ay * ay + az * az
    bb = bx * bx + by * by + bz * bz
    ab = (_b16(ax) * _b16(bx) + _b16(ay) * _b16(by)) + _b16(az) * _b16(bz)
    return jnp.maximum(aa + bb - 2.0 * ab, 0.0)


def _ball_small(sqd, radius, k):
    s, p = sqd.shape
    niota = _iota((1, p), 1)
    pf = float(p)
    dsel = jnp.where(sqd < radius * radius, niota, pf)
    cols = [jnp.min(dsel, axis=1, keepdims=True)]
    for _ in range(k - 1):
        mk = jnp.min(jnp.where(dsel > cols[-1], dsel, BIG),
                     axis=1, keepdims=True)
        cols.append(mk)
    idx = jnp.concatenate(cols, axis=1)
    first = idx[:, 0:1]
    idx = jnp.where(idx >= pf, first, idx)
    idx = jnp.where(idx >= pf, 0.0, idx)
    return idx


def _gather_oh(idxcol, table):
    p = table.shape[0]
    oh = (_iota((1, p), 1) == idxcol).astype(F32)
    return jnp.dot(oh, table, preferred_element_type=F32)


def _sa_small(xyz, feats, nx, radii, nsamples, scales):
    p = xyz.shape[0]
    npoint = nx.shape[0]
    table = jnp.concatenate([xyz, feats], axis=1)
    sqd = _sqd_small(nx, xyz)
    outs = []
    for radius, k, layers in zip(radii, nsamples, scales):
        idx = _ball_small(sqd, radius, k)
        hs = []
        for j in range(k):
            rows = _gather_oh(idx[:, j:j + 1], table)
            hs.append(jnp.concatenate([rows[:, 0:3] - nx, rows[:, 3:]], axis=1))
        h = jnp.concatenate(hs, axis=0)
        h = _mlp_rows(h, layers)
        o = h[0:npoint]
        for j in range(1, k):
            o = jnp.maximum(o, h[j * npoint:(j + 1) * npoint])
        outs.append(o)
    return jnp.concatenate(outs, axis=1)


def _interp3(unknown, known, kn_f):
    sqd = _sqd_small(unknown, known)
    m1 = jnp.min(sqd, axis=1, keepdims=True)
    m2 = jnp.min(jnp.where(sqd > m1, sqd, BIG), axis=1, keepdims=True)
    m3 = jnp.min(jnp.where(sqd > m2, sqd, BIG), axis=1, keepdims=True)
    w = jnp.where(sqd <= m3, 1.0 / (jnp.sqrt(sqd) + 1e-8), 0.0)
    w = w / jnp.sum(w, axis=1, keepdims=True)
    return jnp.dot(w, kn_f, preferred_element_type=F32)


def _mid_body(cfg2, cfg3, nl, args):
    xyz1_ref, f1_ref, nx2_ref, nx3_ref = args[:4]
    wrefs = args[4:4 + 2 * sum(nl)]
    out_ref = args[4 + 2 * sum(nl)]

    def take(off, count):
        return [(wrefs[off + 2 * j][...], wrefs[off + 2 * j + 1][...])
                for j in range(count)]

    o = 0
    sa2_l0 = take(o, nl[0]); o += 2 * nl[0]
    sa2_l1 = take(o, nl[1]); o += 2 * nl[1]
    sa3_l0 = take(o, nl[2]); o += 2 * nl[2]
    sa3_l1 = take(o, nl[3]); o += 2 * nl[3]
    fp2_l = take(o, nl[4]); o += 2 * nl[4]
    fp1_l = take(o, nl[5]); o += 2 * nl[5]

    xyz1 = xyz1_ref[0]
    f1 = f1_ref[0]
    xyz2 = nx2_ref[0]
    xyz3 = nx3_ref[0]
    _, radii2, ns2 = cfg2
    _, radii3, ns3 = cfg3
    f2 = _sa_small(xyz1, f1, xyz2, radii2, ns2, (sa2_l0, sa2_l1))
    f3 = _sa_small(xyz2, f2, xyz3, radii3, ns3, (sa3_l0, sa3_l1))
    h = jnp.concatenate([_interp3(xyz2, xyz3, f3), f2], axis=1)
    f2p = _mlp_rows(h, fp2_l)
    h = jnp.concatenate([_interp3(xyz1, xyz2, f2p), f1], axis=1)
    out_ref[0] = _mlp_rows(h, fp1_l)


def _mid(nx1, f1, nx2, nx3, wlists, cfg2, cfg3, b):
    npoint = nx1.shape[1]
    np2 = nx2.shape[1]
    np3 = nx3.shape[1]
    c1 = f1.shape[-1]
    nl = tuple(len(l) for l in wlists)
    wargs, wspecs = [], []
    for lst in wlists:
        for (w, bias) in lst:
            wargs += [w, bias]
            wspecs += [pl.BlockSpec(w.shape, lambda i: (0, 0)),
                       pl.BlockSpec(bias.shape, lambda i: (0, 0))]
    cout = int(wlists[-1][-1][0].shape[1])
    body = functools.partial(_mid_body, cfg2, cfg3, nl)

    def wrapped(*refs):
        body(refs)

    return pl.pallas_call(
        wrapped,
        grid=(b,),
        in_specs=[
            pl.BlockSpec((1, npoint, 3), lambda i: (i, 0, 0)),
            pl.BlockSpec((1, npoint, c1), lambda i: (i, 0, 0)),
            pl.BlockSpec((1, np2, 3), lambda i: (i, 0, 0)),
            pl.BlockSpec((1, np3, 3), lambda i: (i, 0, 0)),
        ] + wspecs,
        out_specs=[pl.BlockSpec((1, npoint, cout), lambda i: (i, 0, 0))],
        out_shape=[jax.ShapeDtypeStruct((b, npoint, cout), F32)],
        interpret=_INTERPRET,
    )(nx1, f1, nx2, nx3, *wargs)[0]


def _head_body(nl, args):
    pc_ref, xyzt_ref, nxT_ref, f1p_ref = args[:4]
    wa_ref, wb0_ref, b1_ref = args[4:7]
    wrefs = args[7:7 + 2 * nl]
    out_ref = args[7 + 2 * nl]
    layers = [(wrefs[2 * j][...], wrefs[2 * j + 1][...]) for j in range(nl)]
    pc = pc_ref[0]
    tile = pc.shape[0]
    pxr = xyzt_ref[0, 0:1, :]
    pyr = xyzt_ref[0, 1:2, :]
    pzr = xyzt_ref[0, 2:3, :]
    aa = (pxr * pxr + pyr * pyr + pzr * pzr).reshape(tile, 1)
    nxxr = nxT_ref[0, 0:1, :]
    nxyr = nxT_ref[0, 1:2, :]
    nxzr = nxT_ref[0, 2:3, :]
    bb = nxxr * nxxr + nxyr * nxyr + nxzr * nxzr
    ab = jnp.dot(pc[:, 0:3].astype(jnp.bfloat16),
                 nxT_ref[0].astype(jnp.bfloat16),
                 preferred_element_type=F32)
    sqd = jnp.maximum(aa + bb - 2.0 * ab, 0.0)
    m1 = jnp.min(sqd, axis=1, keepdims=True)
    m2 = jnp.min(jnp.where(sqd > m1, sqd, BIG), axis=1, keepdims=True)
    m3 = jnp.min(jnp.where(sqd > m2, sqd, BIG), axis=1, keepdims=True)
    w = jnp.where(sqd <= m3, 1.0 / (jnp.sqrt(sqd) + 1e-8), 0.0)
    w = w / jnp.sum(w, axis=1, keepdims=True)
    t = jnp.dot(f1p_ref[0], wa_ref[...], preferred_element_type=F32)
    h = jnp.dot(w, t, preferred_element_type=F32)
    h = h + jnp.dot(pc, wb0_ref[...], preferred_element_type=F32)
    h = jnp.maximum(h + b1_ref[...], 0.0)
    for j in range(nl - 1):
        wj, bias = layers[j]
        h = jnp.maximum(jnp.dot(h, wj, preferred_element_type=F32) + bias, 0.0)
    wj, bias = layers[nl - 1]
    out_ref[0] = jnp.dot(h, wj, preferred_element_type=F32) + bias


def _head(pointcloud, xyzT, nxT, f1p, first, layers, b, n, tile):
    ch = pointcloud.shape[-1]
    npoint = nxT.shape[2]
    c1 = f1p.shape[-1]
    wargs, wspecs = [], []
    for arr in first:
        wargs.append(arr)
        wspecs.append(pl.BlockSpec(arr.shape, lambda i, t: (0, 0)))
    for (w, bias) in layers:
        wargs += [w, bias]
        wspecs += [pl.BlockSpec(w.shape, lambda i, t: (0, 0)),
                   pl.BlockSpec(bias.shape, lambda i, t: (0, 0))]
    ncls = int(layers[-1][0].shape[1])
    body = functools.partial(_head_body, len(layers))

    def wrapped(*refs):
        body(refs)

    return pl.pallas_call(
        wrapped,
        grid=(b, n // tile),
        in_specs=[
            pl.BlockSpec((1, tile, ch), lambda i, t: (i, t, 0)),
            pl.BlockSpec((1, 3, tile), lambda i, t: (i, 0, t)),
            pl.BlockSpec((1, 3, npoint), lambda i, t: (i, 0, 0)),
            pl.BlockSpec((1, npoint, c1), lambda i, t: (i, 0, 0)),
        ] + wspecs,
        out_specs=[pl.BlockSpec((1, tile, ncls), lambda i, t: (i, t, 0))],
        out_shape=[jax.ShapeDtypeStruct((b, n, ncls), F32)],
        interpret=_INTERPRET,
    )(pointcloud, xyzT, nxT, f1p, *wargs)[0]


def _fold(layers):
    out = []
    for (w, g, bias) in layers:
        out.append(((w * (g * _BN_INV)[:, None]).T, bias.reshape(1, -1)))
    return out


def kernel(pointcloud, params):
    b, n, ch = pointcloud.shape
    xyzT = jnp.transpose(pointcloud[..., :3], (0, 2, 1))

    sa_cfg = [(128, (1.0, 3.0), (2, 8)),
              (128, (2.0, 4.0), (2, 8)),
              (64, (3.0, 6.0), (4, 8))]

    np1, radii1, ns1 = sa_cfg[0]
    nx1T = _fps1(xyzT, n, b, np1)
    nx1 = jnp.transpose(nx1T, (0, 2, 1))
    i0, i1 = _bq1(xyzT, nx1T, nx1, n, b, np1, radii1, ns1)

    sa1_layers = [_fold(s) for s in params['sa'][0]]
    f1 = _group1(pointcloud, nx1, i0, i1, sa1_layers, n, b, np1, ns1)

    nx2, nx3 = _fps23(nx1T, b, sa_cfg[1][0], sa_cfg[2][0])
    wlists = ([_fold(s) for s in params['sa'][1]]
              + [_fold(s) for s in params['sa'][2]]
              + [_fold(params['fp'][2]), _fold(params['fp'][1])])
    f1p = _mid(nx1, f1, nx2, nx3, wlists, sa_cfg[1], sa_cfg[2], b)

    fc = params['fc']
    head_layers = _fold(params['fp'][0]) + [
        ((fc['W1'] * (fc['g1'] * _BN_INV)[:, None]).T, fc['b1'].reshape(1, -1)),
        (fc['W2'].T, fc['b2'].reshape(1, -1)),
        (fc['W3'].T, fc['b3'].reshape(1, -1)),
    ]
    w1, b1 = head_layers[0]
    wa = w1[0:32]
    wb0 = jnp.concatenate([jnp.zeros((3, w1.shape[1]), F32), w1[32:]], axis=0)
    return _head(pointcloud, xyzT, nx1T, f1p, (wa, wb0, b1), head_layers[1:],
                 b, n, 4096)


# speedup vs baseline: 37.2142x; 1.0166x over previous
import functools
import numpy as np
import jax
import jax.numpy as jnp
from jax.experimental import pallas as pl
from jax.experimental.pallas import tpu as pltpu

_INTERPRET = False

_BN_INV = 1.0 / np.sqrt(1.0 + 1e-5)
F32 = jnp.float32
BIG = 3.0e7


def _iota(shape, dim):
    return jax.lax.broadcasted_iota(jnp.int32, shape, dim).astype(F32)


def _b16(x):
    return x.astype(jnp.bfloat16).astype(F32)


def _fps1_body(npoint, xyzT_ref, nx_ref):
    px = xyzT_ref[:, 0, :]
    py = xyzT_ref[:, 1, :]
    pz = xyzT_ref[:, 2, :]
    nxx, nxy, nxz = _fps_batched(px, py, pz, npoint)
    nx_ref[:, 0, :] = nxx
    nx_ref[:, 1, :] = nxy
    nx_ref[:, 2, :] = nxz


def _fps1(xyzT, n, b, npoint):
    body = functools.partial(_fps1_body, npoint)
    return pl.pallas_call(
        body,
        grid=(1,),
        in_specs=[pl.BlockSpec((b, 3, n), lambda i: (0, 0, 0))],
        out_specs=[pl.BlockSpec((b, 3, npoint), lambda i: (0, 0, 0))],
        out_shape=[jax.ShapeDtypeStruct((b, 3, npoint), F32)],
        interpret=_INTERPRET,
    )(xyzT)[0]


def _bq1_body(n, npoint, radii, nsamples, xyzT_ref, nxT_ref, nx_ref,
              i0_ref, i1_ref):
    px = xyzT_ref[0, 0:1, :]
    py = xyzT_ref[0, 1:2, :]
    pz = xyzT_ref[0, 2:3, :]
    nxx = nxT_ref[0, 0:1, :].reshape(npoint, 1)
    nxy = nxT_ref[0, 1:2, :].reshape(npoint, 1)
    nxz = nxT_ref[0, 2:3, :].reshape(npoint, 1)
    aa = nxx * nxx + nxy * nxy + nxz * nxz
    bb = px * px + py * py + pz * pz
    ab = jnp.dot(nx_ref[0].astype(jnp.bfloat16),
                 xyzT_ref[0].astype(jnp.bfloat16),
                 preferred_element_type=F32)
    sqd = jnp.maximum(aa + bb - 2.0 * ab, 0.0)
    niota = _iota((1, n), 1)
    nf = float(n)
    for (radius, k, out_ref) in ((radii[0], nsamples[0], i0_ref),
                                 (radii[1], nsamples[1], i1_ref)):
        dsel = jnp.where(sqd < radius * radius, niota, nf)
        cols = [jnp.min(dsel, axis=1, keepdims=True)]
        for _ in range(k - 1):
            mk = jnp.min(jnp.where(dsel > cols[-1], dsel, BIG),
                         axis=1, keepdims=True)
            cols.append(mk)
        idx = jnp.concatenate(cols, axis=1)
        first = idx[:, 0:1]
        idx = jnp.where(idx >= nf, first, idx)
        idx = jnp.where(idx >= nf, 0.0, idx)
        out_ref[0] = idx.astype(jnp.int32)


def _bq1(xyzT, nxT, nx, n, b, npoint, radii, nsamples):
    body = functools.partial(_bq1_body, n, npoint, radii, nsamples)
    return pl.pallas_call(
        body,
        grid=(b,),
        in_specs=[
            pl.BlockSpec((1, 3, n), lambda i: (i, 0, 0)),
            pl.BlockSpec((1, 3, npoint), lambda i: (i, 0, 0)),
            pl.BlockSpec((1, npoint, 3), lambda i: (i, 0, 0)),
        ],
        out_specs=[
            pl.BlockSpec((1, npoint, nsamples[0]), lambda i: (i, 0, 0)),
            pl.BlockSpec((1, npoint, nsamples[1]), lambda i: (i, 0, 0)),
        ],
        out_shape=[
            jax.ShapeDtypeStruct((b, npoint, nsamples[0]), jnp.int32),
            jax.ShapeDtypeStruct((b, npoint, nsamples[1]), jnp.int32),
        ],
        interpret=_INTERPRET,
    )(xyzT, nxT, nx)


def _mlp_rows(h, layers):
    for (w, bias) in layers:
        h = jnp.maximum(jnp.dot(h, w, preferred_element_type=F32) + bias, 0.0)
    return h


def _group1_body(n, npoint, nsamples, nlayers, args):
    pc_ref, nx_ref, i0_ref, i1_ref = args[:4]
    wrefs = args[4:4 + 2 * nlayers[0] + 2 * nlayers[1]]
    f_ref = args[4 + 2 * nlayers[0] + 2 * nlayers[1]]
    g0_ref, g1_ref = args[-2:]
    k0, k1 = nsamples

    def gather(s, _):
        for k in range(k0):
            i = i0_ref[0, s, k]
            g0_ref[pl.ds(k * npoint + s, 1), :] = pc_ref[0, pl.ds(i, 1), :]
        for k in range(k1):
            i = i1_ref[0, s, k]
            g1_ref[pl.ds(k * npoint + s, 1), :] = pc_ref[0, pl.ds(i, 1), :]
        return 0

    jax.lax.fori_loop(0, npoint, gather, 0)
    nx = nx_ref[0]
    outs = []
    woff = 0
    for (k, g_ref, nl) in ((k0, g0_ref, nlayers[0]), (k1, g1_ref, nlayers[1])):
        layers = [(wrefs[woff + 2 * j][...], wrefs[woff + 2 * j + 1][...])
                  for j in range(nl)]
        woff += 2 * nl
        g = g_ref[...]
        cent = jnp.concatenate([nx] * k, axis=0)
        h = jnp.concatenate([g[:, 0:3] - cent, g[:, 3:]], axis=1)
        h = _mlp_rows(h, layers)
        o = h[0:npoint]
        for j in range(1, k):
            o = jnp.maximum(o, h[j * npoint:(j + 1) * npoint])
        outs.append(o)
    f_ref[0] = jnp.concatenate(outs, axis=1)


def _group1(pointcloud, nx, i0, i1, sa_layers, n, b, npoint, nsamples):
    ch = pointcloud.shape[-1]
    nlayers = (len(sa_layers[0]), len(sa_layers[1]))
    wargs, wspecs = [], []
    for scale in sa_layers:
        for (w, bias) in scale:
            wargs += [w, bias]
            wspecs += [pl.BlockSpec(w.shape, lambda i: (0, 0)),
                       pl.BlockSpec(bias.shape, lambda i: (0, 0))]
    cout = sum(int(s[-1][0].shape[1]) for s in sa_layers)
    body = functools.partial(_group1_body, n, npoint, nsamples, nlayers)

    def wrapped(*refs):
        body(refs)

    return pl.pallas_call(
        wrapped,
        grid=(b,),
        in_specs=[
            pl.BlockSpec((1, n, ch), lambda i: (i, 0, 0)),
            pl.BlockSpec((1, npoint, 3), lambda i: (i, 0, 0)),
            pl.BlockSpec((1, npoint, nsamples[0]), lambda i: (i, 0, 0),
                         memory_space=pltpu.SMEM),
            pl.BlockSpec((1, npoint, nsamples[1]), lambda i: (i, 0, 0),
                         memory_space=pltpu.SMEM),
        ] + wspecs,
        out_specs=[pl.BlockSpec((1, npoint, cout), lambda i: (i, 0, 0))],
        out_shape=[jax.ShapeDtypeStruct((b, npoint, cout), F32)],
        scratch_shapes=[pltpu.VMEM((nsamples[0] * npoint, ch), F32),
                        pltpu.VMEM((nsamples[1] * npoint, ch), F32)],
        interpret=_INTERPRET,
    )(pointcloud, nx, i0, i1, *wargs)[0]


def _fps_batched(px, py, pz, npoint):
    b, p = px.shape
    niota = _iota((1, p), 1)
    citer = _iota((1, npoint), 1)

    def step(s, carry):
        dists, far, nxx, nxy, nxz = carry
        sel = (niota == far)
        cx = jnp.sum(jnp.where(sel, px, 0.0), axis=1, keepdims=True)
        cy = jnp.sum(jnp.where(sel, py, 0.0), axis=1, keepdims=True)
        cz = jnp.sum(jnp.where(sel, pz, 0.0), axis=1, keepdims=True)
        scol = (citer == s.astype(F32))
        nxx = jnp.where(scol, cx, nxx)
        nxy = jnp.where(scol, cy, nxy)
        nxz = jnp.where(scol, cz, nxz)
        dx = px - cx
        dy = py - cy
        dz = pz - cz
        d = dx * dx + dy * dy + dz * dz
        dists = jnp.minimum(dists, d)
        m = jnp.max(dists, axis=1, keepdims=True)
        far = jnp.min(jnp.where(dists == m, niota, BIG), axis=1, keepdims=True)
        return dists, far, nxx, nxy, nxz

    init = (jnp.full((b, p), 1e10, F32), jnp.zeros((b, 1), F32),
            jnp.zeros((b, npoint), F32), jnp.zeros((b, npoint), F32),
            jnp.zeros((b, npoint), F32))
    _, _, nxx, nxy, nxz = jax.lax.fori_loop(0, npoint, step, init)
    return nxx, nxy, nxz


def _fps23_body(np2, np3, xyz1T_ref, nx2_ref, nx3_ref):
    px = xyz1T_ref[:, 0, :]
    py = xyz1T_ref[:, 1, :]
    pz = xyz1T_ref[:, 2, :]
    n2x, n2y, n2z = _fps_batched(px, py, pz, np2)
    nx2_ref[:, 0, :] = n2x
    nx2_ref[:, 1, :] = n2y
    nx2_ref[:, 2, :] = n2z
    n3x, n3y, n3z = _fps_batched(n2x, n2y, n2z, np3)
    nx3_ref[:, 0, :] = n3x
    nx3_ref[:, 1, :] = n3y
    nx3_ref[:, 2, :] = n3z


def _fps23(xyz1T, b, np2, np3):
    p = xyz1T.shape[2]
    body = functools.partial(_fps23_body, np2, np3)
    nx2T, nx3T = pl.pallas_call(
        body,
        grid=(1,),
        in_specs=[pl.BlockSpec((b, 3, p), lambda i: (0, 0, 0))],
        out_specs=[
            pl.BlockSpec((b, 3, np2), lambda i: (0, 0, 0)),
            pl.BlockSpec((b, 3, np3), lambda i: (0, 0, 0)),
        ],
        out_shape=[
            jax.ShapeDtypeStruct((b, 3, np2), F32),
            jax.ShapeDtypeStruct((b, 3, np3), F32),
        ],
        interpret=_INTERPRET,
    )(xyz1T)
    return jnp.transpose(nx2T, (0, 2, 1)), jnp.transpose(nx3T, (0, 2, 1))


def _sqd_small(a, bpts):
    p = bpts.shape[0]
    ax, ay, az = a[:, 0:1], a[:, 1:2], a[:, 2:3]
    bx = bpts[:, 0:1].reshape(1, p)
    by = bpts[:, 1:2].reshape(1, p)
    bz = bpts[:, 2:3].reshape(1, p)
    aa = ax * ax + ay * ay + az * az
    bb = bx * bx + by * by + bz * bz
    ab = (_b16(ax) * _b16(bx) + _b16(ay) * _b16(by)) + _b16(az) * _b16(bz)
    return jnp.maximum(aa + bb - 2.0 * ab, 0.0)


def _ball_small(sqd, radius, k):
    s, p = sqd.shape
    niota = _iota((1, p), 1)
    pf = float(p)
    dsel = jnp.where(sqd < radius * radius, niota, pf)
    cols = [jnp.min(dsel, axis=1, keepdims=True)]
    for _ in range(k - 1):
        mk = jnp.min(jnp.where(dsel > cols[-1], dsel, BIG),
                     axis=1, keepdims=True)
        cols.append(mk)
    idx = jnp.concatenate(cols, axis=1)
    first = idx[:, 0:1]
    idx = jnp.where(idx >= pf, first, idx)
    idx = jnp.where(idx >= pf, 0.0, idx)
    return idx


def _gather_oh(idxcol, table):
    p = table.shape[0]
    oh = (_iota((1, p), 1) == idxcol).astype(F32)
    return jnp.dot(oh, table, preferred_element_type=F32)


def _sa_small(xyz, feats, nx, radii, nsamples, scales):
    p = xyz.shape[0]
    npoint = nx.shape[0]
    table = jnp.concatenate([xyz, feats], axis=1)
    sqd = _sqd_small(nx, xyz)
    outs = []
    for radius, k, layers in zip(radii, nsamples, scales):
        idx = _ball_small(sqd, radius, k)
        hs = []
        for j in range(k):
            rows = _gather_oh(idx[:, j:j + 1], table)
            hs.append(jnp.concatenate([rows[:, 0:3] - nx, rows[:, 3:]], axis=1))
        h = jnp.concatenate(hs, axis=0)
        h = _mlp_rows(h, layers)
        o = h[0:npoint]
        for j in range(1, k):
            o = jnp.maximum(o, h[j * npoint:(j + 1) * npoint])
        outs.append(o)
    return jnp.concatenate(outs, axis=1)


def _interp3(unknown, known, kn_f):
    sqd = _sqd_small(unknown, known)
    m1 = jnp.min(sqd, axis=1, keepdims=True)
    m2 = jnp.min(jnp.where(sqd > m1, sqd, BIG), axis=1, keepdims=True)
    m3 = jnp.min(jnp.where(sqd > m2, sqd, BIG), axis=1, keepdims=True)
    w = jnp.where(sqd <= m3, 1.0 / (jnp.sqrt(sqd) + 1e-8), 0.0)
    w = w / jnp.sum(w, axis=1, keepdims=True)
    return jnp.dot(w, kn_f, preferred_element_type=F32)


def _mid_body(cfg2, cfg3, nl, args):
    xyz1_ref, f1_ref, nx2_ref, nx3_ref = args[:4]
    wrefs = args[4:4 + 2 * sum(nl)]
    out_ref = args[4 + 2 * sum(nl)]

    def take(off, count):
        return [(wrefs[off + 2 * j][...], wrefs[off + 2 * j + 1][...])
                for j in range(count)]

    o = 0
    sa2_l0 = take(o, nl[0]); o += 2 * nl[0]
    sa2_l1 = take(o, nl[1]); o += 2 * nl[1]
    sa3_l0 = take(o, nl[2]); o += 2 * nl[2]
    sa3_l1 = take(o, nl[3]); o += 2 * nl[3]
    fp2_l = take(o, nl[4]); o += 2 * nl[4]
    fp1_l = take(o, nl[5]); o += 2 * nl[5]

    xyz1 = xyz1_ref[0]
    f1 = f1_ref[0]
    xyz2 = nx2_ref[0]
    xyz3 = nx3_ref[0]
    _, radii2, ns2 = cfg2
    _, radii3, ns3 = cfg3
    f2 = _sa_small(xyz1, f1, xyz2, radii2, ns2, (sa2_l0, sa2_l1))
    f3 = _sa_small(xyz2, f2, xyz3, radii3, ns3, (sa3_l0, sa3_l1))
    h = jnp.concatenate([_interp3(xyz2, xyz3, f3), f2], axis=1)
    f2p = _mlp_rows(h, fp2_l)
    h = jnp.concatenate([_interp3(xyz1, xyz2, f2p), f1], axis=1)
    out_ref[0] = _mlp_rows(h, fp1_l)


def _mid(nx1, f1, nx2, nx3, wlists, cfg2, cfg3, b):
    npoint = nx1.shape[1]
    np2 = nx2.shape[1]
    np3 = nx3.shape[1]
    c1 = f1.shape[-1]
    nl = tuple(len(l) for l in wlists)
    wargs, wspecs = [], []
    for lst in wlists:
        for (w, bias) in lst:
            wargs += [w, bias]
            wspecs += [pl.BlockSpec(w.shape, lambda i: (0, 0)),
                       pl.BlockSpec(bias.shape, lambda i: (0, 0))]
    cout = int(wlists[-1][-1][0].shape[1])
    body = functools.partial(_mid_body, cfg2, cfg3, nl)

    def wrapped(*refs):
        body(refs)

    return pl.pallas_call(
        wrapped,
        grid=(b,),
        in_specs=[
            pl.BlockSpec((1, npoint, 3), lambda i: (i, 0, 0)),
            pl.BlockSpec((1, npoint, c1), lambda i: (i, 0, 0)),
            pl.BlockSpec((1, np2, 3), lambda i: (i, 0, 0)),
            pl.BlockSpec((1, np3, 3), lambda i: (i, 0, 0)),
        ] + wspecs,
        out_specs=[pl.BlockSpec((1, npoint, cout), lambda i: (i, 0, 0))],
        out_shape=[jax.ShapeDtypeStruct((b, npoint, cout), F32)],
        interpret=_INTERPRET,
    )(nx1, f1, nx2, nx3, *wargs)[0]


def _head_body(nl, args):
    pc_ref, xyzt_ref, nxT_ref, f1p_ref = args[:4]
    wa_ref, wb0_ref, b1_ref = args[4:7]
    wrefs = args[7:7 + 2 * nl]
    out_ref = args[7 + 2 * nl]
    layers = [(wrefs[2 * j][...], wrefs[2 * j + 1][...]) for j in range(nl)]
    pc = pc_ref[0]
    tile = pc.shape[0]
    pxr = xyzt_ref[0, 0:1, :]
    pyr = xyzt_ref[0, 1:2, :]
    pzr = xyzt_ref[0, 2:3, :]
    aa = (pxr * pxr + pyr * pyr + pzr * pzr).reshape(tile, 1)
    nxxr = nxT_ref[0, 0:1, :]
    nxyr = nxT_ref[0, 1:2, :]
    nxzr = nxT_ref[0, 2:3, :]
    bb = nxxr * nxxr + nxyr * nxyr + nxzr * nxzr
    ab = jnp.dot(pc[:, 0:3].astype(jnp.bfloat16),
                 nxT_ref[0].astype(jnp.bfloat16),
                 preferred_element_type=F32)
    sqd = jnp.maximum(aa + bb - 2.0 * ab, 0.0)
    m1 = jnp.min(sqd, axis=1, keepdims=True)
    m2 = jnp.min(jnp.where(sqd > m1, sqd, BIG), axis=1, keepdims=True)
    m3 = jnp.min(jnp.where(sqd > m2, sqd, BIG), axis=1, keepdims=True)
    w = jnp.where(sqd <= m3, 1.0 / (jnp.sqrt(sqd) + 1e-8), 0.0)
    w = w / jnp.sum(w, axis=1, keepdims=True)
    t = jnp.dot(f1p_ref[0], wa_ref[...], preferred_element_type=F32)
    h = jnp.dot(w, t, preferred_element_type=F32)
    h = h + jnp.dot(pc, wb0_ref[...], preferred_element_type=F32)
    h = jnp.maximum(h + b1_ref[...], 0.0)
    for j in range(nl - 1):
        wj, bias = layers[j]
        h = jnp.maximum(jnp.dot(h, wj, preferred_element_type=F32) + bias, 0.0)
    wj, bias = layers[nl - 1]
    out_ref[0] = jnp.dot(h, wj, preferred_element_type=F32) + bias


def _head(pointcloud, xyzT, nxT, f1p, first, layers, b, n, tile):
    ch = pointcloud.shape[-1]
    npoint = nxT.shape[2]
    c1 = f1p.shape[-1]
    wargs, wspecs = [], []
    for arr in first:
        wargs.append(arr)
        wspecs.append(pl.BlockSpec(arr.shape, lambda i, t: (0, 0)))
    for (w, bias) in layers:
        wargs += [w, bias]
        wspecs += [pl.BlockSpec(w.shape, lambda i, t: (0, 0)),
                   pl.BlockSpec(bias.shape, lambda i, t: (0, 0))]
    ncls = int(layers[-1][0].shape[1])
    body = functools.partial(_head_body, len(layers))

    def wrapped(*refs):
        body(refs)

    return pl.pallas_call(
        wrapped,
        grid=(b, n // tile),
        in_specs=[
            pl.BlockSpec((1, tile, ch), lambda i, t: (i, t, 0)),
            pl.BlockSpec((1, 3, tile), lambda i, t: (i, 0, t)),
            pl.BlockSpec((1, 3, npoint), lambda i, t: (i, 0, 0)),
            pl.BlockSpec((1, npoint, c1), lambda i, t: (i, 0, 0)),
        ] + wspecs,
        out_specs=[pl.BlockSpec((1, tile, ncls), lambda i, t: (i, t, 0))],
        out_shape=[jax.ShapeDtypeStruct((b, n, ncls), F32)],
        interpret=_INTERPRET,
    )(pointcloud, xyzT, nxT, f1p, *wargs)[0]


def _fold(layers):
    out = []
    for (w, g, bias) in layers:
        out.append(((w * (g * _BN_INV)[:, None]).T, bias.reshape(1, -1)))
    return out


def kernel(pointcloud, params):
    b, n, ch = pointcloud.shape
    xyzT = jnp.transpose(pointcloud[..., :3], (0, 2, 1))

    sa_cfg = [(128, (1.0, 3.0), (2, 8)),
              (128, (2.0, 4.0), (2, 8)),
              (64, (3.0, 6.0), (4, 8))]

    np1, radii1, ns1 = sa_cfg[0]
    nx1T = _fps1(xyzT, n, b, np1)
    nx1 = jnp.transpose(nx1T, (0, 2, 1))
    i0, i1 = _bq1(xyzT, nx1T, nx1, n, b, np1, radii1, ns1)

    sa1_layers = [_fold(s) for s in params['sa'][0]]
    f1 = _group1(pointcloud, nx1, i0, i1, sa1_layers, n, b, np1, ns1)

    nx2, nx3 = _fps23(nx1T, b, sa_cfg[1][0], sa_cfg[2][0])
    wlists = ([_fold(s) for s in params['sa'][1]]
              + [_fold(s) for s in params['sa'][2]]
              + [_fold(params['fp'][2]), _fold(params['fp'][1])])
    f1p = _mid(nx1, f1, nx2, nx3, wlists, sa_cfg[1], sa_cfg[2], b)

    fc = params['fc']
    head_layers = _fold(params['fp'][0]) + [
        ((fc['W1'] * (fc['g1'] * _BN_INV)[:, None]).T, fc['b1'].reshape(1, -1)),
        (fc['W2'].T, fc['b2'].reshape(1, -1)),
        (fc['W3'].T, fc['b3'].reshape(1, -1)),
    ]
    w1, b1 = head_layers[0]
    wa = w1[0:32]
    wb0 = jnp.concatenate([jnp.zeros((3, w1.shape[1]), F32), w1[32:]], axis=0)
    return _head(pointcloud, xyzT, nx1T, f1p, (wa, wb0, b1), head_layers[1:],
                 b, n, 8192)
